# baseline (speedup 1.0000x reference)
import jax
import jax.numpy as jnp
from jax import lax
from jax.experimental import pallas as pl
from jax.experimental.pallas import tpu as pltpu
from jax.experimental.pallas import tpu_sc as plsc

N = 100000
E = 1600000
D = 32
H = 16
G = 64
LANES = 16
NC = 2
NS = 16
NW = NC * NS

EP = 1622016
ROWS = EP // 128
RPW = ROWS // NW
RPT = ROWS // NS
K = 4
CHUNK = K * 128
KD = 18

NN = 100352
NSEG = NN // NS
NPW = NN // NW
PC = 1568
PROWS = NN * H // 128

BR = 1792
NBLK = PROWS // BR

_mesh = plsc.VectorSubcoreMesh(core_axis_name="core", subcore_axis_name="subcore")
_sc_params = pltpu.CompilerParams(use_tc_tiling_on_sc=False)
_sc_params_nolayout = pltpu.CompilerParams(use_tc_tiling_on_sc=False,
                                           needs_layout_passes=False)

_DN = lax.GatherDimensionNumbers(
    offset_dims=(), collapsed_slice_dims=(0,), start_index_map=(0,))


def _bcast(vec, l):
    return lax.gather(vec, jnp.full((LANES, 1), l, jnp.int32), _DN, (1,),
                      mode=lax.GatherScatterMode.PROMISE_IN_BOUNDS)


def _rsqrt16(d):
    i = plsc.bitcast(d, jnp.int32)
    y = plsc.bitcast(jnp.int32(0x5F3759DF) - (i >> 1), jnp.float32)
    for _ in range(3):
        y = y * (1.5 - 0.5 * d * y * y)
    return y


def _deg_body(dst_hbm, ew_hbm, out0_hbm, out1_hbm, dstv0, ewv0, dstv1, ewv1,
              zbuf, acc, seml0, seml1, sems0, sems1):
    cid = lax.axis_index("core")
    sid = lax.axis_index("subcore")
    wid = sid * NC + cid

    @pl.loop(0, NSEG, step=LANES)
    def _(i):
        zbuf[pl.ds(i, LANES)] = jnp.zeros((LANES,), jnp.float32)

    pltpu.sync_copy(zbuf, acc.at[pl.ds(sid * NSEG, NSEG)])
    plsc.subcore_barrier()

    base = wid * RPW

    @pl.loop(0, RPW, step=2 * KD)
    def _(r):
        la = [pltpu.async_copy(dst_hbm.at[pl.ds(base + r, KD)], dstv0, seml0),
              pltpu.async_copy(ew_hbm.at[pl.ds(base + r, KD)], ewv0, seml0)]
        lb = [pltpu.async_copy(dst_hbm.at[pl.ds(base + r + KD, KD)], dstv1,
                               seml1),
              pltpu.async_copy(ew_hbm.at[pl.ds(base + r + KD, KD)], ewv1,
                               seml1)]
        for cp in la:
            cp.wait()
        sa = [pltpu.async_copy(ewv0.at[j], acc.at[dstv0.at[j]], sems0,
                               add=True) for j in range(KD)]
        for cp in lb:
            cp.wait()
        sb = [pltpu.async_copy(ewv1.at[j], acc.at[dstv1.at[j]], sems1,
                               add=True) for j in range(KD)]
        for cp in sa:
            cp.wait()
        for cp in sb:
            cp.wait()

    plsc.subcore_barrier()
    pltpu.sync_copy(acc.at[pl.ds(sid * NSEG, NSEG)], zbuf)

    @pl.when(cid == 0)
    def _():
        pltpu.sync_copy(zbuf, out0_hbm.at[pl.ds(sid * NSEG, NSEG)])

    @pl.when(cid == 1)
    def _():
        pltpu.sync_copy(zbuf, out1_hbm.at[pl.ds(sid * NSEG, NSEG)])


_deg_call = pl.kernel(
    _deg_body,
    out_type=[jax.ShapeDtypeStruct((NN,), jnp.float32),
              jax.ShapeDtypeStruct((NN,), jnp.float32)],
    mesh=_mesh,
    scratch_types=[
        pltpu.VMEM((KD, 128), jnp.int32),
        pltpu.VMEM((KD, 128), jnp.float32),
        pltpu.VMEM((KD, 128), jnp.int32),
        pltpu.VMEM((KD, 128), jnp.float32),
        pltpu.VMEM((NSEG,), jnp.float32),
        pltpu.VMEM_SHARED((NN,), jnp.float32),
        pltpu.SemaphoreType.DMA,
        pltpu.SemaphoreType.DMA,
        pltpu.SemaphoreType.DMA,
        pltpu.SemaphoreType.DMA,
    ],
    compiler_params=_sc_params,
)


def _prep_body(x_hbm, w1_hbm, p0_hbm, p1_hbm, tlo_hbm, thi_hbm, dis_hbm,
               xb, w1b, p0b, p1b, lob, hib, dsb):
    cid = lax.axis_index("core")
    sid = lax.axis_index("subcore")
    wid = sid * NC + cid
    nbase = wid * NPW

    pltpu.sync_copy(w1_hbm, w1b)
    w1v = [[w1b[pl.ds(32 * k + 16 * h_, LANES)] for h_ in range(2)]
           for k in range(3)]

    @pl.loop(0, NPW, step=PC)
    def _(c):
        node0 = nbase + c
        pltpu.sync_copy(x_hbm.at[pl.ds(node0 * 3, PC * 3)], xb)
        pltpu.sync_copy(p0_hbm.at[pl.ds(node0, PC)], p0b)
        pltpu.sync_copy(p1_hbm.at[pl.ds(node0, PC)], p1b)

        @pl.loop(0, PC, step=LANES)
        def _(g):
            d = p0b[pl.ds(g, LANES)] + p1b[pl.ds(g, LANES)] + 1.0
            y = _rsqrt16(d)
            xv = [xb[pl.ds(g * 3, LANES)],
                  xb[pl.ds(g * 3 + 16, LANES)],
                  xb[pl.ds(g * 3 + 32, LANES)]]
            for l in range(LANES):
                a = [_bcast(xv[(3 * l + k) // 16], (3 * l + k) % 16)
                     for k in range(3)]
                db = _bcast(y, l)
                rl = (a[0] * w1v[0][0] + a[1] * w1v[1][0]
                      + a[2] * w1v[2][0]) * db
                rh = (a[0] * w1v[0][1] + a[1] * w1v[1][1]
                      + a[2] * w1v[2][1]) * db
                lob[g + l] = rl
                hib[g + l] = rh
                dsb[g + l] = db

        pltpu.sync_copy(lob, tlo_hbm.at[pl.ds(node0, PC)])
        pltpu.sync_copy(hib, thi_hbm.at[pl.ds(node0, PC)])
        pltpu.sync_copy(dsb, dis_hbm.at[pl.ds(node0, PC)])


_prep_call = pl.kernel(
    _prep_body,
    out_type=[jax.ShapeDtypeStruct((NN, H), jnp.float32),
              jax.ShapeDtypeStruct((NN, H), jnp.float32),
              jax.ShapeDtypeStruct((NN, H), jnp.float32)],
    mesh=_mesh,
    scratch_types=[
        pltpu.VMEM((PC * 3,), jnp.float32),
        pltpu.VMEM((96,), jnp.float32),
        pltpu.VMEM((PC,), jnp.float32),
        pltpu.VMEM((PC,), jnp.float32),
        pltpu.VMEM((PC, H), jnp.float32),
        pltpu.VMEM((PC, H), jnp.float32),
        pltpu.VMEM((PC, H), jnp.float32),
    ],
    compiler_params=_sc_params_nolayout,
)


def _scale_rows(rows, ewv):
    for j in range(K):
        @pl.loop(0, 128, step=LANES)
        def _(e):
            w16 = ewv[j, pl.ds(e, LANES)]
            for l in range(LANES):
                idx = j * 128 + e + l
                rows[idx] = rows[idx] * _bcast(w16, l)


def _agg_half(src_hbm, dst_hbm, ew_hbm, table, out, bufs, acc, sid):
    (srcv0, dstv0, ewv0, rows0, srcv1, dstv1, ewv1, rows1,
     seml0, seml1, semg0, semg1, sems0, sems1) = bufs
    rows = rows0
    @pl.loop(0, CHUNK)
    def _(i):
        rows[i] = jnp.zeros((LANES,), jnp.float32)

    abase = sid * NSEG

    @pl.loop(0, NSEG - CHUNK + 1, step=CHUNK)
    def _(i):
        pltpu.sync_copy(rows, acc.at[pl.ds(abase + i, CHUNK)])

    rem = NSEG % CHUNK
    pltpu.sync_copy(rows.at[pl.ds(0, rem)],
                    acc.at[pl.ds(abase + NSEG - rem, rem)])
    plsc.subcore_barrier()

    base = sid * RPT

    def _linears(r, srcv, dstv, ewv, sem):
        return [pltpu.async_copy(src_hbm.at[pl.ds(r, K)], srcv, sem),
                pltpu.async_copy(dst_hbm.at[pl.ds(r, K)], dstv, sem),
                pltpu.async_copy(ew_hbm.at[pl.ds(r, K)], ewv, sem)]

    def _gathers(srcv, rows, sem):
        return [pltpu.async_copy(table.at[srcv.at[j]],
                                 rows.at[pl.ds(j * 128, 128)], sem)
                for j in range(K)]

    def _scatters(rows, dstv, sem):
        return [pltpu.async_copy(rows.at[pl.ds(j * 128, 128)],
                                 acc.at[dstv.at[j]], sem, add=True)
                for j in range(K)]

    def _drain(cps):
        for cp in cps:
            cp.wait()

    @pl.loop(0, RPT, step=2 * K)
    def _(r):
        l0 = _linears(base + r, srcv0, dstv0, ewv0, seml0)
        l1 = _linears(base + r + K, srcv1, dstv1, ewv1, seml1)
        _drain(l0)
        g0 = _gathers(srcv0, rows0, semg0)
        _drain(l1)
        g1 = _gathers(srcv1, rows1, semg1)
        _drain(g0)
        _scale_rows(rows0, ewv0)
        s0 = _scatters(rows0, dstv0, sems0)
        _drain(g1)
        _scale_rows(rows1, ewv1)
        s1 = _scatters(rows1, dstv1, sems1)
        _drain(s0)
        _drain(s1)

    plsc.subcore_barrier()
    @pl.loop(0, NSEG - CHUNK + 1, step=CHUNK)
    def _(i):
        pltpu.sync_copy(acc.at[pl.ds(abase + i, CHUNK)], rows)
        pltpu.sync_copy(rows, out.at[pl.ds(abase + i, CHUNK)])

    pltpu.sync_copy(acc.at[pl.ds(abase + NSEG - rem, rem)],
                    rows.at[pl.ds(0, rem)])
    pltpu.sync_copy(rows.at[pl.ds(0, rem)],
                    out.at[pl.ds(abase + NSEG - rem, rem)])


def _agg_body(src_hbm, dst_hbm, ew_hbm, tlo_hbm, thi_hbm, olo_hbm, ohi_hbm,
              *bufs_acc):
    acc = bufs_acc[8]
    bufs = bufs_acc[:8] + bufs_acc[9:]
    cid = lax.axis_index("core")
    sid = lax.axis_index("subcore")

    @pl.when(cid == 0)
    def _():
        _agg_half(src_hbm, dst_hbm, ew_hbm, tlo_hbm, olo_hbm, bufs, acc, sid)

    @pl.when(cid == 1)
    def _():
        _agg_half(src_hbm, dst_hbm, ew_hbm, thi_hbm, ohi_hbm, bufs, acc, sid)


_agg_call = pl.kernel(
    _agg_body,
    out_type=[jax.ShapeDtypeStruct((NN, H), jnp.float32),
              jax.ShapeDtypeStruct((NN, H), jnp.float32)],
    mesh=_mesh,
    scratch_types=[
        pltpu.VMEM((K, 128), jnp.int32),
        pltpu.VMEM((K, 128), jnp.int32),
        pltpu.VMEM((K, 128), jnp.float32),
        pltpu.VMEM((CHUNK, H), jnp.float32),
        pltpu.VMEM((K, 128), jnp.int32),
        pltpu.VMEM((K, 128), jnp.int32),
        pltpu.VMEM((K, 128), jnp.float32),
        pltpu.VMEM((CHUNK, H), jnp.float32),
        pltpu.VMEM_SHARED((NN, H), jnp.float32),
        pltpu.SemaphoreType.DMA,
        pltpu.SemaphoreType.DMA,
        pltpu.SemaphoreType.DMA,
        pltpu.SemaphoreType.DMA,
        pltpu.SemaphoreType.DMA,
        pltpu.SemaphoreType.DMA,
    ],
    compiler_params=_sc_params,
)


def _mid_body(alo_ref, ahi_ref, tlo_ref, thi_ref, dis_ref, b1lo_ref, b1hi_ref,
              kll_ref, khl_ref, klh_ref, khh_ref, lo_ref, hi_ref):
    disp = dis_ref[...]
    hlo = jnp.maximum(disp * (alo_ref[...] + tlo_ref[...]) + b1lo_ref[...], 0.0)
    hhi = jnp.maximum(disp * (ahi_ref[...] + thi_ref[...]) + b1hi_ref[...], 0.0)
    h2lo = (jnp.dot(hlo, kll_ref[...], preferred_element_type=jnp.float32)
            + jnp.dot(hhi, khl_ref[...], preferred_element_type=jnp.float32))
    h2hi = (jnp.dot(hlo, klh_ref[...], preferred_element_type=jnp.float32)
            + jnp.dot(hhi, khh_ref[...], preferred_element_type=jnp.float32))
    lo_ref[...] = h2lo * disp
    hi_ref[...] = h2hi * disp


_mid_call = pl.pallas_call(
    _mid_body,
    grid=(NBLK,),
    in_specs=[
        pl.BlockSpec((BR, 128), lambda i: (i, 0)),
        pl.BlockSpec((BR, 128), lambda i: (i, 0)),
        pl.BlockSpec((BR, 128), lambda i: (i, 0)),
        pl.BlockSpec((BR, 128), lambda i: (i, 0)),
        pl.BlockSpec((BR, 128), lambda i: (i, 0)),
        pl.BlockSpec((1, 128), lambda i: (0, 0)),
        pl.BlockSpec((1, 128), lambda i: (0, 0)),
        pl.BlockSpec((128, 128), lambda i: (0, 0)),
        pl.BlockSpec((128, 128), lambda i: (0, 0)),
        pl.BlockSpec((128, 128), lambda i: (0, 0)),
        pl.BlockSpec((128, 128), lambda i: (0, 0)),
    ],
    out_specs=[
        pl.BlockSpec((BR, 128), lambda i: (i, 0)),
        pl.BlockSpec((BR, 128), lambda i: (i, 0)),
    ],
    out_shape=[jax.ShapeDtypeStruct((PROWS, 128), jnp.float32),
               jax.ShapeDtypeStruct((PROWS, 128), jnp.float32)],
)


def _fin_body(alo_ref, ahi_ref, tlo_ref, thi_ref, dis_ref, b2lo_ref, b2hi_ref,
              batch_ref, wo_ref, bo_ref, wb_ref, bb_ref,
              orange_ref, blue_ref, slo, shi, cnt):
    i = pl.program_id(0)

    @pl.when(i == 0)
    def _():
        slo[...] = jnp.zeros_like(slo)
        shi[...] = jnp.zeros_like(shi)
        cnt[...] = jnp.zeros_like(cnt)

    disp = dis_ref[...]
    hlo = jnp.maximum(disp * (alo_ref[...] + tlo_ref[...]) + b2lo_ref[...], 0.0)
    hhi = jnp.maximum(disp * (ahi_ref[...] + thi_ref[...]) + b2hi_ref[...], 0.0)
    bv = batch_ref[...]
    gids = lax.broadcasted_iota(jnp.int32, (G, BR), 0)
    for j in range(8):
        mask = (bv[j:j + 1, :] == gids).astype(jnp.float32)
        slo[...] += jnp.dot(mask, hlo[:, 16 * j:16 * j + 16],
                            preferred_element_type=jnp.float32)
        shi[...] += jnp.dot(mask, hhi[:, 16 * j:16 * j + 16],
                            preferred_element_type=jnp.float32)
        cnt[...] += jnp.sum(mask, axis=1, keepdims=True)

    @pl.when(i == NBLK - 1)
    def _():
        c = jnp.maximum(cnt[...], 1.0)
        glo = slo[...] / c
        ghi = shi[...] / c
        wo = wo_ref[...]
        wb = wb_ref[...]
        orange_ref[...] = jax.nn.sigmoid(
            jnp.dot(glo, wo[:16, :], preferred_element_type=jnp.float32)
            + jnp.dot(ghi, wo[16:, :], preferred_element_type=jnp.float32)
            + bo_ref[...])
        blue_ref[...] = jax.nn.sigmoid(
            jnp.dot(glo, wb[:16, :], preferred_element_type=jnp.float32)
            + jnp.dot(ghi, wb[16:, :], preferred_element_type=jnp.float32)
            + bb_ref[...])


_fin_call = pl.pallas_call(
    _fin_body,
    grid=(NBLK,),
    in_specs=[
        pl.BlockSpec((BR, 128), lambda i: (i, 0)),
        pl.BlockSpec((BR, 128), lambda i: (i, 0)),
        pl.BlockSpec((BR, 128), lambda i: (i, 0)),
        pl.BlockSpec((BR, 128), lambda i: (i, 0)),
        pl.BlockSpec((BR, 128), lambda i: (i, 0)),
        pl.BlockSpec((1, 128), lambda i: (0, 0)),
        pl.BlockSpec((1, 128), lambda i: (0, 0)),
        pl.BlockSpec((8, BR), lambda i: (0, i)),
        pl.BlockSpec((D, 1), lambda i: (0, 0)),
        pl.BlockSpec((1, 1), lambda i: (0, 0)),
        pl.BlockSpec((D, 1), lambda i: (0, 0)),
        pl.BlockSpec((1, 1), lambda i: (0, 0)),
    ],
    out_specs=[
        pl.BlockSpec((G, 1), lambda i: (0, 0)),
        pl.BlockSpec((G, 1), lambda i: (0, 0)),
    ],
    out_shape=[jax.ShapeDtypeStruct((G, 1), jnp.float32),
               jax.ShapeDtypeStruct((G, 1), jnp.float32)],
    scratch_shapes=[pltpu.VMEM((G, H), jnp.float32),
                    pltpu.VMEM((G, H), jnp.float32),
                    pltpu.VMEM((G, 1), jnp.float32)],
)


def kernel(x, edge_index, edge_weight, batch, W1, b1, W2, b2, Wo, bo, Wb, bb):
    pad = EP - E
    src = jnp.concatenate(
        [edge_index[0], jnp.zeros((pad,), jnp.int32)]).reshape(ROWS, 128)
    dst = jnp.concatenate(
        [edge_index[1], jnp.zeros((pad,), jnp.int32)]).reshape(ROWS, 128)
    ew = jnp.concatenate(
        [edge_weight, jnp.zeros((pad,), jnp.float32)]).reshape(ROWS, 128)

    xflat = jnp.concatenate(
        [x.reshape(-1), jnp.zeros(((NN - N) * 3,), jnp.float32)])
    w1flat = W1.reshape(-1)

    p0, p1 = _deg_call(dst, ew)
    tlo1, thi1, dis16 = _prep_call(xflat, w1flat, p0, p1)
    alo1, ahi1 = _agg_call(src, dst, ew, tlo1, thi1)

    disp = dis16.reshape(PROWS, 128)
    eye8 = jnp.eye(8, dtype=jnp.float32)
    kll = jnp.kron(eye8, W2[:16, :16])
    khl = jnp.kron(eye8, W2[16:, :16])
    klh = jnp.kron(eye8, W2[:16, 16:])
    khh = jnp.kron(eye8, W2[16:, 16:])
    b1lo = jnp.tile(b1[:16], 8).reshape(1, 128)
    b1hi = jnp.tile(b1[16:], 8).reshape(1, 128)
    b2lo = jnp.tile(b2[:16], 8).reshape(1, 128)
    b2hi = jnp.tile(b2[16:], 8).reshape(1, 128)

    lo2p, hi2p = _mid_call(alo1.reshape(PROWS, 128), ahi1.reshape(PROWS, 128),
                           tlo1.reshape(PROWS, 128), thi1.reshape(PROWS, 128),
                           disp, b1lo, b1hi, kll, khl, klh, khh)

    alo2, ahi2 = _agg_call(src, dst, ew,
                           lo2p.reshape(NN, H), hi2p.reshape(NN, H))

    batchp = jnp.concatenate(
        [batch, jnp.full((NN - N,), -1, jnp.int32)]).reshape(PROWS, 8).T

    orange, blue = _fin_call(alo2.reshape(PROWS, 128), ahi2.reshape(PROWS, 128),
                             lo2p, hi2p, disp, b2lo, b2hi, batchp,
                             Wo, bo.reshape(1, 1), Wb, bb.reshape(1, 1))
    return orange, blue


# speedup vs baseline: 1.3218x; 1.3218x over previous
import jax
import jax.numpy as jnp
from jax import lax
from jax.experimental import pallas as pl
from jax.experimental.pallas import tpu as pltpu
from jax.experimental.pallas import tpu_sc as plsc

N = 100000
E = 1600000
D = 32
H = 16
G = 64
LANES = 16
NC = 2
NS = 16
NW = NC * NS

EP = 1605632
ROWS = EP // 128
RPW = ROWS // NW
RPT = ROWS // NS
K = 4
CHUNK = K * 128
KD = 14

NN = 100352
NSEG = NN // NS
NPW = NN // NW
PC = 1568
PROWS = NN * H // 128

BR = 1792
NBLK = PROWS // BR

_mesh = plsc.VectorSubcoreMesh(core_axis_name="core", subcore_axis_name="subcore")
_sc_params = pltpu.CompilerParams(use_tc_tiling_on_sc=False)
_sc_params_nolayout = pltpu.CompilerParams(use_tc_tiling_on_sc=False,
                                           needs_layout_passes=False)

_DN = lax.GatherDimensionNumbers(
    offset_dims=(), collapsed_slice_dims=(0,), start_index_map=(0,))


def _bcast(vec, l):
    return lax.gather(vec, jnp.full((LANES, 1), l, jnp.int32), _DN, (1,),
                      mode=lax.GatherScatterMode.PROMISE_IN_BOUNDS)


def _rsqrt16(d):
    i = plsc.bitcast(d, jnp.int32)
    y = plsc.bitcast(jnp.int32(0x5F3759DF) - (i >> 1), jnp.float32)
    for _ in range(3):
        y = y * (1.5 - 0.5 * d * y * y)
    return y


def _deg_body(dst_hbm, ew_hbm, out0_hbm, out1_hbm, dstv0, ewv0, dstv1, ewv1,
              zbuf, acc, seml0, seml1, sems0, sems1):
    cid = lax.axis_index("core")
    sid = lax.axis_index("subcore")
    wid = sid * NC + cid

    @pl.loop(0, NSEG, step=LANES)
    def _(i):
        zbuf[pl.ds(i, LANES)] = jnp.zeros((LANES,), jnp.float32)

    pltpu.sync_copy(zbuf, acc.at[pl.ds(sid * NSEG, NSEG)])
    plsc.subcore_barrier()

    base = wid * RPW

    @pl.loop(0, RPW, step=2 * KD)
    def _(r):
        la = [pltpu.async_copy(dst_hbm.at[pl.ds(base + r, KD)], dstv0, seml0),
              pltpu.async_copy(ew_hbm.at[pl.ds(base + r, KD)], ewv0, seml0)]
        lb = [pltpu.async_copy(dst_hbm.at[pl.ds(base + r + KD, KD)], dstv1,
                               seml1),
              pltpu.async_copy(ew_hbm.at[pl.ds(base + r + KD, KD)], ewv1,
                               seml1)]
        for cp in la:
            cp.wait()
        sa = [pltpu.async_copy(ewv0.at[j], acc.at[dstv0.at[j]], sems0,
                               add=True) for j in range(KD)]
        for cp in lb:
            cp.wait()
        sb = [pltpu.async_copy(ewv1.at[j], acc.at[dstv1.at[j]], sems1,
                               add=True) for j in range(KD)]
        for cp in sa:
            cp.wait()
        for cp in sb:
            cp.wait()

    plsc.subcore_barrier()
    pltpu.sync_copy(acc.at[pl.ds(sid * NSEG, NSEG)], zbuf)

    @pl.when(cid == 0)
    def _():
        pltpu.sync_copy(zbuf, out0_hbm.at[pl.ds(sid * NSEG, NSEG)])

    @pl.when(cid == 1)
    def _():
        pltpu.sync_copy(zbuf, out1_hbm.at[pl.ds(sid * NSEG, NSEG)])


_deg_call = pl.kernel(
    _deg_body,
    out_type=[jax.ShapeDtypeStruct((NN,), jnp.float32),
              jax.ShapeDtypeStruct((NN,), jnp.float32)],
    mesh=_mesh,
    scratch_types=[
        pltpu.VMEM((KD, 128), jnp.int32),
        pltpu.VMEM((KD, 128), jnp.float32),
        pltpu.VMEM((KD, 128), jnp.int32),
        pltpu.VMEM((KD, 128), jnp.float32),
        pltpu.VMEM((NSEG,), jnp.float32),
        pltpu.VMEM_SHARED((NN,), jnp.float32),
        pltpu.SemaphoreType.DMA,
        pltpu.SemaphoreType.DMA,
        pltpu.SemaphoreType.DMA,
        pltpu.SemaphoreType.DMA,
    ],
    compiler_params=_sc_params,
)


def _prep_body(x_hbm, w1_hbm, p0_hbm, p1_hbm, tlo_hbm, thi_hbm, dis_hbm,
               xb, w1b, p0b, p1b, lob, hib, dsb):
    cid = lax.axis_index("core")
    sid = lax.axis_index("subcore")
    wid = sid * NC + cid
    nbase = wid * NPW

    pltpu.sync_copy(w1_hbm, w1b)
    w1v = [[w1b[pl.ds(32 * k + 16 * h_, LANES)] for h_ in range(2)]
           for k in range(3)]

    @pl.loop(0, NPW, step=PC)
    def _(c):
        node0 = nbase + c
        pltpu.sync_copy(x_hbm.at[pl.ds(node0 * 3, PC * 3)], xb)
        pltpu.sync_copy(p0_hbm.at[pl.ds(node0, PC)], p0b)
        pltpu.sync_copy(p1_hbm.at[pl.ds(node0, PC)], p1b)

        @pl.loop(0, PC, step=LANES)
        def _(g):
            d = p0b[pl.ds(g, LANES)] + p1b[pl.ds(g, LANES)] + 1.0
            y = _rsqrt16(d)
            xv = [xb[pl.ds(g * 3, LANES)],
                  xb[pl.ds(g * 3 + 16, LANES)],
                  xb[pl.ds(g * 3 + 32, LANES)]]
            for l in range(LANES):
                a = [_bcast(xv[(3 * l + k) // 16], (3 * l + k) % 16)
                     for k in range(3)]
                db = _bcast(y, l)
                rl = (a[0] * w1v[0][0] + a[1] * w1v[1][0]
                      + a[2] * w1v[2][0]) * db
                rh = (a[0] * w1v[0][1] + a[1] * w1v[1][1]
                      + a[2] * w1v[2][1]) * db
                lob[g + l] = rl
                hib[g + l] = rh
                dsb[g + l] = db

        pltpu.sync_copy(lob, tlo_hbm.at[pl.ds(node0, PC)])
        pltpu.sync_copy(hib, thi_hbm.at[pl.ds(node0, PC)])
        pltpu.sync_copy(dsb, dis_hbm.at[pl.ds(node0, PC)])


_prep_call = pl.kernel(
    _prep_body,
    out_type=[jax.ShapeDtypeStruct((NN, H), jnp.float32),
              jax.ShapeDtypeStruct((NN, H), jnp.float32),
              jax.ShapeDtypeStruct((NN, H), jnp.float32)],
    mesh=_mesh,
    scratch_types=[
        pltpu.VMEM((PC * 3,), jnp.float32),
        pltpu.VMEM((96,), jnp.float32),
        pltpu.VMEM((PC,), jnp.float32),
        pltpu.VMEM((PC,), jnp.float32),
        pltpu.VMEM((PC, H), jnp.float32),
        pltpu.VMEM((PC, H), jnp.float32),
        pltpu.VMEM((PC, H), jnp.float32),
    ],
    compiler_params=_sc_params_nolayout,
)


def _scale_rows(rows, ewv):
    for j in range(K):
        @pl.loop(0, 128, step=LANES)
        def _(e):
            w16 = ewv[j, pl.ds(e, LANES)]
            for l in range(LANES):
                idx = j * 128 + e + l
                rows[idx] = rows[idx] * _bcast(w16, l)


def _agg_half(src_hbm, dst_hbm, ew_hbm, table, out, bufs, acc, sid):
    (srcv0, dstv0, ewv0, rows0, srcv1, dstv1, ewv1, rows1,
     seml0, seml1, semg0, semg1, sems0, sems1) = bufs
    rows = rows0
    @pl.loop(0, CHUNK)
    def _(i):
        rows[i] = jnp.zeros((LANES,), jnp.float32)

    abase = sid * NSEG

    @pl.loop(0, NSEG - CHUNK + 1, step=CHUNK)
    def _(i):
        pltpu.sync_copy(rows, acc.at[pl.ds(abase + i, CHUNK)])

    rem = NSEG % CHUNK
    pltpu.sync_copy(rows.at[pl.ds(0, rem)],
                    acc.at[pl.ds(abase + NSEG - rem, rem)])
    plsc.subcore_barrier()

    base = sid * RPT

    def _linears(r, srcv, dstv, ewv, sem):
        return [pltpu.async_copy(src_hbm.at[pl.ds(r, K)], srcv, sem),
                pltpu.async_copy(dst_hbm.at[pl.ds(r, K)], dstv, sem),
                pltpu.async_copy(ew_hbm.at[pl.ds(r, K)], ewv, sem)]

    def _gathers(srcv, rows, sem):
        return [pltpu.async_copy(table.at[srcv.at[j]],
                                 rows.at[pl.ds(j * 128, 128)], sem)
                for j in range(K)]

    def _scatters(rows, dstv, sem):
        return [pltpu.async_copy(rows.at[pl.ds(j * 128, 128)],
                                 acc.at[dstv.at[j]], sem, add=True)
                for j in range(K)]

    def _drain(cps):
        for cp in cps:
            cp.wait()

    @pl.loop(0, RPT, step=2 * K)
    def _(r):
        l0 = _linears(base + r, srcv0, dstv0, ewv0, seml0)
        l1 = _linears(base + r + K, srcv1, dstv1, ewv1, seml1)
        _drain(l0)
        g0 = _gathers(srcv0, rows0, semg0)
        _drain(l1)
        g1 = _gathers(srcv1, rows1, semg1)
        _drain(g0)
        _scale_rows(rows0, ewv0)
        s0 = _scatters(rows0, dstv0, sems0)
        _drain(g1)
        _scale_rows(rows1, ewv1)
        s1 = _scatters(rows1, dstv1, sems1)
        _drain(s0)
        _drain(s1)

    plsc.subcore_barrier()
    @pl.loop(0, NSEG - CHUNK + 1, step=CHUNK)
    def _(i):
        pltpu.sync_copy(acc.at[pl.ds(abase + i, CHUNK)], rows)
        pltpu.sync_copy(rows, out.at[pl.ds(abase + i, CHUNK)])

    pltpu.sync_copy(acc.at[pl.ds(abase + NSEG - rem, rem)],
                    rows.at[pl.ds(0, rem)])
    pltpu.sync_copy(rows.at[pl.ds(0, rem)],
                    out.at[pl.ds(abase + NSEG - rem, rem)])


def _agg_body(src_hbm, dst_hbm, ew_hbm, tlo_hbm, thi_hbm, olo_hbm, ohi_hbm,
              *bufs_acc):
    acc = bufs_acc[8]
    bufs = bufs_acc[:8] + bufs_acc[9:]
    cid = lax.axis_index("core")
    sid = lax.axis_index("subcore")

    @pl.when(cid == 0)
    def _():
        _agg_half(src_hbm, dst_hbm, ew_hbm, tlo_hbm, olo_hbm, bufs, acc, sid)

    @pl.when(cid == 1)
    def _():
        _agg_half(src_hbm, dst_hbm, ew_hbm, thi_hbm, ohi_hbm, bufs, acc, sid)


_agg_call = pl.kernel(
    _agg_body,
    out_type=[jax.ShapeDtypeStruct((NN, H), jnp.float32),
              jax.ShapeDtypeStruct((NN, H), jnp.float32)],
    mesh=_mesh,
    scratch_types=[
        pltpu.VMEM((K, 128), jnp.int32),
        pltpu.VMEM((K, 128), jnp.int32),
        pltpu.VMEM((K, 128), jnp.float32),
        pltpu.VMEM((CHUNK, H), jnp.float32),
        pltpu.VMEM((K, 128), jnp.int32),
        pltpu.VMEM((K, 128), jnp.int32),
        pltpu.VMEM((K, 128), jnp.float32),
        pltpu.VMEM((CHUNK, H), jnp.float32),
        pltpu.VMEM_SHARED((NN, H), jnp.float32),
        pltpu.SemaphoreType.DMA,
        pltpu.SemaphoreType.DMA,
        pltpu.SemaphoreType.DMA,
        pltpu.SemaphoreType.DMA,
        pltpu.SemaphoreType.DMA,
        pltpu.SemaphoreType.DMA,
    ],
    compiler_params=_sc_params,
)


def _mid_body(alo_ref, ahi_ref, tlo_ref, thi_ref, dis_ref, b1lo_ref, b1hi_ref,
              kll_ref, khl_ref, klh_ref, khh_ref, lo_ref, hi_ref):
    disp = dis_ref[...]
    hlo = jnp.maximum(disp * (alo_ref[...] + tlo_ref[...]) + b1lo_ref[...], 0.0)
    hhi = jnp.maximum(disp * (ahi_ref[...] + thi_ref[...]) + b1hi_ref[...], 0.0)
    h2lo = (jnp.dot(hlo, kll_ref[...], preferred_element_type=jnp.float32)
            + jnp.dot(hhi, khl_ref[...], preferred_element_type=jnp.float32))
    h2hi = (jnp.dot(hlo, klh_ref[...], preferred_element_type=jnp.float32)
            + jnp.dot(hhi, khh_ref[...], preferred_element_type=jnp.float32))
    lo_ref[...] = h2lo * disp
    hi_ref[...] = h2hi * disp


_mid_call = pl.pallas_call(
    _mid_body,
    grid=(NBLK,),
    in_specs=[
        pl.BlockSpec((BR, 128), lambda i: (i, 0)),
        pl.BlockSpec((BR, 128), lambda i: (i, 0)),
        pl.BlockSpec((BR, 128), lambda i: (i, 0)),
        pl.BlockSpec((BR, 128), lambda i: (i, 0)),
        pl.BlockSpec((BR, 128), lambda i: (i, 0)),
        pl.BlockSpec((1, 128), lambda i: (0, 0)),
        pl.BlockSpec((1, 128), lambda i: (0, 0)),
        pl.BlockSpec((128, 128), lambda i: (0, 0)),
        pl.BlockSpec((128, 128), lambda i: (0, 0)),
        pl.BlockSpec((128, 128), lambda i: (0, 0)),
        pl.BlockSpec((128, 128), lambda i: (0, 0)),
    ],
    out_specs=[
        pl.BlockSpec((BR, 128), lambda i: (i, 0)),
        pl.BlockSpec((BR, 128), lambda i: (i, 0)),
    ],
    out_shape=[jax.ShapeDtypeStruct((PROWS, 128), jnp.float32),
               jax.ShapeDtypeStruct((PROWS, 128), jnp.float32)],
)


def _fin_body(alo_ref, ahi_ref, tlo_ref, thi_ref, dis_ref, b2lo_ref, b2hi_ref,
              batch_ref, wo_ref, bo_ref, wb_ref, bb_ref,
              orange_ref, blue_ref, slo, shi, cnt):
    i = pl.program_id(0)

    @pl.when(i == 0)
    def _():
        slo[...] = jnp.zeros_like(slo)
        shi[...] = jnp.zeros_like(shi)
        cnt[...] = jnp.zeros_like(cnt)

    disp = dis_ref[...]
    hlo = jnp.maximum(disp * (alo_ref[...] + tlo_ref[...]) + b2lo_ref[...], 0.0)
    hhi = jnp.maximum(disp * (ahi_ref[...] + thi_ref[...]) + b2hi_ref[...], 0.0)
    bv = batch_ref[...]
    gids = lax.broadcasted_iota(jnp.int32, (G, BR), 0)
    for j in range(8):
        mask = (bv[j:j + 1, :] == gids).astype(jnp.float32)
        slo[...] += jnp.dot(mask, hlo[:, 16 * j:16 * j + 16],
                            preferred_element_type=jnp.float32)
        shi[...] += jnp.dot(mask, hhi[:, 16 * j:16 * j + 16],
                            preferred_element_type=jnp.float32)
        cnt[...] += jnp.sum(mask, axis=1, keepdims=True)

    @pl.when(i == NBLK - 1)
    def _():
        c = jnp.maximum(cnt[...], 1.0)
        glo = slo[...] / c
        ghi = shi[...] / c
        wo = wo_ref[...]
        wb = wb_ref[...]
        orange_ref[...] = jax.nn.sigmoid(
            jnp.dot(glo, wo[:16, :], preferred_element_type=jnp.float32)
            + jnp.dot(ghi, wo[16:, :], preferred_element_type=jnp.float32)
            + bo_ref[...])
        blue_ref[...] = jax.nn.sigmoid(
            jnp.dot(glo, wb[:16, :], preferred_element_type=jnp.float32)
            + jnp.dot(ghi, wb[16:, :], preferred_element_type=jnp.float32)
            + bb_ref[...])


_fin_call = pl.pallas_call(
    _fin_body,
    grid=(NBLK,),
    in_specs=[
        pl.BlockSpec((BR, 128), lambda i: (i, 0)),
        pl.BlockSpec((BR, 128), lambda i: (i, 0)),
        pl.BlockSpec((BR, 128), lambda i: (i, 0)),
        pl.BlockSpec((BR, 128), lambda i: (i, 0)),
        pl.BlockSpec((BR, 128), lambda i: (i, 0)),
        pl.BlockSpec((1, 128), lambda i: (0, 0)),
        pl.BlockSpec((1, 128), lambda i: (0, 0)),
        pl.BlockSpec((8, BR), lambda i: (0, i)),
        pl.BlockSpec((D, 1), lambda i: (0, 0)),
        pl.BlockSpec((1, 1), lambda i: (0, 0)),
        pl.BlockSpec((D, 1), lambda i: (0, 0)),
        pl.BlockSpec((1, 1), lambda i: (0, 0)),
    ],
    out_specs=[
        pl.BlockSpec((G, 1), lambda i: (0, 0)),
        pl.BlockSpec((G, 1), lambda i: (0, 0)),
    ],
    out_shape=[jax.ShapeDtypeStruct((G, 1), jnp.float32),
               jax.ShapeDtypeStruct((G, 1), jnp.float32)],
    scratch_shapes=[pltpu.VMEM((G, H), jnp.float32),
                    pltpu.VMEM((G, H), jnp.float32),
                    pltpu.VMEM((G, 1), jnp.float32)],
)


def kernel(x, edge_index, edge_weight, batch, W1, b1, W2, b2, Wo, bo, Wb, bb):
    pad = EP - E
    padidx = (jnp.arange(pad, dtype=jnp.int32) * 61) % N
    src = jnp.concatenate([edge_index[0], padidx]).reshape(ROWS, 128)
    dst = jnp.concatenate([edge_index[1], padidx]).reshape(ROWS, 128)
    ew = jnp.concatenate(
        [edge_weight, jnp.zeros((pad,), jnp.float32)]).reshape(ROWS, 128)

    xflat = jnp.concatenate(
        [x.reshape(-1), jnp.zeros(((NN - N) * 3,), jnp.float32)])
    w1flat = W1.reshape(-1)

    p0, p1 = _deg_call(dst, ew)
    tlo1, thi1, dis16 = _prep_call(xflat, w1flat, p0, p1)
    alo1, ahi1 = _agg_call(src, dst, ew, tlo1, thi1)

    disp = dis16.reshape(PROWS, 128)
    eye8 = jnp.eye(8, dtype=jnp.float32)
    kll = jnp.kron(eye8, W2[:16, :16])
    khl = jnp.kron(eye8, W2[16:, :16])
    klh = jnp.kron(eye8, W2[:16, 16:])
    khh = jnp.kron(eye8, W2[16:, 16:])
    b1lo = jnp.tile(b1[:16], 8).reshape(1, 128)
    b1hi = jnp.tile(b1[16:], 8).reshape(1, 128)
    b2lo = jnp.tile(b2[:16], 8).reshape(1, 128)
    b2hi = jnp.tile(b2[16:], 8).reshape(1, 128)

    lo2p, hi2p = _mid_call(alo1.reshape(PROWS, 128), ahi1.reshape(PROWS, 128),
                           tlo1.reshape(PROWS, 128), thi1.reshape(PROWS, 128),
                           disp, b1lo, b1hi, kll, khl, klh, khh)

    alo2, ahi2 = _agg_call(src, dst, ew,
                           lo2p.reshape(NN, H), hi2p.reshape(NN, H))

    batchp = jnp.concatenate(
        [batch, jnp.full((NN - N,), -1, jnp.int32)]).reshape(PROWS, 8).T

    orange, blue = _fin_call(alo2.reshape(PROWS, 128), ahi2.reshape(PROWS, 128),
                             lo2p, hi2p, disp, b2lo, b2hi, batchp,
                             Wo, bo.reshape(1, 1), Wb, bb.reshape(1, 1))
    return orange, blue


# speedup vs baseline: 1.3718x; 1.0378x over previous
import jax
import jax.numpy as jnp
from jax import lax
from jax.experimental import pallas as pl
from jax.experimental.pallas import tpu as pltpu
from jax.experimental.pallas import tpu_sc as plsc

N = 100000
E = 1600000
D = 32
H = 16
G = 64
LANES = 16
NC = 2
NS = 16
NW = NC * NS

EP = 1622016
ROWS = EP // 128
RPW = ROWS // NW
RPT = ROWS // NS
K = 4
CHUNK = K * 128
KD = 18

NN = 100352
NSEG = NN // NS
NPW = NN // NW
PC = 1568
PROWS = NN * H // 128

BR = 1792
NBLK = PROWS // BR

_mesh = plsc.VectorSubcoreMesh(core_axis_name="core", subcore_axis_name="subcore")
_sc_params = pltpu.CompilerParams(use_tc_tiling_on_sc=False)
_sc_params_nolayout = pltpu.CompilerParams(use_tc_tiling_on_sc=False,
                                           needs_layout_passes=False)

_DN = lax.GatherDimensionNumbers(
    offset_dims=(), collapsed_slice_dims=(0,), start_index_map=(0,))


def _bcast(vec, l):
    return lax.gather(vec, jnp.full((LANES, 1), l, jnp.int32), _DN, (1,),
                      mode=lax.GatherScatterMode.PROMISE_IN_BOUNDS)


def _rsqrt16(d):
    i = plsc.bitcast(d, jnp.int32)
    y = plsc.bitcast(jnp.int32(0x5F3759DF) - (i >> 1), jnp.float32)
    for _ in range(3):
        y = y * (1.5 - 0.5 * d * y * y)
    return y


def _deg_body(dst_hbm, ew_hbm, out0_hbm, out1_hbm, dstv0, ewv0, dstv1, ewv1,
              zbuf, acc, seml0, seml1, sems0, sems1):
    cid = lax.axis_index("core")
    sid = lax.axis_index("subcore")
    wid = sid * NC + cid

    @pl.loop(0, NSEG, step=LANES)
    def _(i):
        zbuf[pl.ds(i, LANES)] = jnp.zeros((LANES,), jnp.float32)

    pltpu.sync_copy(zbuf, acc.at[pl.ds(sid * NSEG, NSEG)])
    plsc.subcore_barrier()

    base = wid * RPW

    @pl.loop(0, RPW, step=2 * KD)
    def _(r):
        la = [pltpu.async_copy(dst_hbm.at[pl.ds(base + r, KD)], dstv0, seml0),
              pltpu.async_copy(ew_hbm.at[pl.ds(base + r, KD)], ewv0, seml0)]
        lb = [pltpu.async_copy(dst_hbm.at[pl.ds(base + r + KD, KD)], dstv1,
                               seml1),
              pltpu.async_copy(ew_hbm.at[pl.ds(base + r + KD, KD)], ewv1,
                               seml1)]
        for cp in la:
            cp.wait()
        sa = [pltpu.async_copy(ewv0.at[j], acc.at[dstv0.at[j]], sems0,
                               add=True) for j in range(KD)]
        for cp in lb:
            cp.wait()
        sb = [pltpu.async_copy(ewv1.at[j], acc.at[dstv1.at[j]], sems1,
                               add=True) for j in range(KD)]
        for cp in sa:
            cp.wait()
        for cp in sb:
            cp.wait()

    plsc.subcore_barrier()
    pltpu.sync_copy(acc.at[pl.ds(sid * NSEG, NSEG)], zbuf)

    @pl.when(cid == 0)
    def _():
        pltpu.sync_copy(zbuf, out0_hbm.at[pl.ds(sid * NSEG, NSEG)])

    @pl.when(cid == 1)
    def _():
        pltpu.sync_copy(zbuf, out1_hbm.at[pl.ds(sid * NSEG, NSEG)])


_deg_call = pl.kernel(
    _deg_body,
    out_type=[jax.ShapeDtypeStruct((NN,), jnp.float32),
              jax.ShapeDtypeStruct((NN,), jnp.float32)],
    mesh=_mesh,
    scratch_types=[
        pltpu.VMEM((KD, 128), jnp.int32),
        pltpu.VMEM((KD, 128), jnp.float32),
        pltpu.VMEM((KD, 128), jnp.int32),
        pltpu.VMEM((KD, 128), jnp.float32),
        pltpu.VMEM((NSEG,), jnp.float32),
        pltpu.VMEM_SHARED((NN,), jnp.float32),
        pltpu.SemaphoreType.DMA,
        pltpu.SemaphoreType.DMA,
        pltpu.SemaphoreType.DMA,
        pltpu.SemaphoreType.DMA,
    ],
    compiler_params=_sc_params,
)


def _prep_body(x_hbm, w1_hbm, p0_hbm, p1_hbm, tlo_hbm, thi_hbm, dis_hbm,
               xb, w1b, p0b, p1b, lob, hib, dsb):
    cid = lax.axis_index("core")
    sid = lax.axis_index("subcore")
    wid = sid * NC + cid
    nbase = wid * NPW

    pltpu.sync_copy(w1_hbm, w1b)
    w1v = [[w1b[pl.ds(32 * k + 16 * h_, LANES)] for h_ in range(2)]
           for k in range(3)]

    @pl.loop(0, NPW, step=PC)
    def _(c):
        node0 = nbase + c
        pltpu.sync_copy(x_hbm.at[pl.ds(node0 * 3, PC * 3)], xb)
        pltpu.sync_copy(p0_hbm.at[pl.ds(node0, PC)], p0b)
        pltpu.sync_copy(p1_hbm.at[pl.ds(node0, PC)], p1b)

        @pl.loop(0, PC, step=LANES)
        def _(g):
            d = p0b[pl.ds(g, LANES)] + p1b[pl.ds(g, LANES)] + 1.0
            y = _rsqrt16(d)
            xv = [xb[pl.ds(g * 3, LANES)],
                  xb[pl.ds(g * 3 + 16, LANES)],
                  xb[pl.ds(g * 3 + 32, LANES)]]
            for l in range(LANES):
                a = [_bcast(xv[(3 * l + k) // 16], (3 * l + k) % 16)
                     for k in range(3)]
                db = _bcast(y, l)
                rl = (a[0] * w1v[0][0] + a[1] * w1v[1][0]
                      + a[2] * w1v[2][0]) * db
                rh = (a[0] * w1v[0][1] + a[1] * w1v[1][1]
                      + a[2] * w1v[2][1]) * db
                lob[g + l] = rl
                hib[g + l] = rh
                dsb[g + l] = db

        pltpu.sync_copy(lob, tlo_hbm.at[pl.ds(node0, PC)])
        pltpu.sync_copy(hib, thi_hbm.at[pl.ds(node0, PC)])
        pltpu.sync_copy(dsb, dis_hbm.at[pl.ds(node0, PC)])


_prep_call = pl.kernel(
    _prep_body,
    out_type=[jax.ShapeDtypeStruct((NN, H), jnp.float32),
              jax.ShapeDtypeStruct((NN, H), jnp.float32),
              jax.ShapeDtypeStruct((NN, H), jnp.float32)],
    mesh=_mesh,
    scratch_types=[
        pltpu.VMEM((PC * 3,), jnp.float32),
        pltpu.VMEM((96,), jnp.float32),
        pltpu.VMEM((PC,), jnp.float32),
        pltpu.VMEM((PC,), jnp.float32),
        pltpu.VMEM((PC, H), jnp.float32),
        pltpu.VMEM((PC, H), jnp.float32),
        pltpu.VMEM((PC, H), jnp.float32),
    ],
    compiler_params=_sc_params_nolayout,
)


def _scale_rows(rows, ewv):
    for j in range(K):
        @pl.loop(0, 128, step=LANES)
        def _(e):
            w16 = ewv[j, pl.ds(e, LANES)]
            for l in range(LANES):
                idx = j * 128 + e + l
                rows[idx] = rows[idx] * _bcast(w16, l)


def _agg_half(src_hbm, dst_hbm, ew_hbm, table, out, bufs, acc, sid):
    (srcv0, dstv0, ewv0, rows0, srcv1, dstv1, ewv1, rows1,
     srcv2, dstv2, ewv2, rows2,
     seml0, seml1, seml2, semg0, semg1, semg2, sems0, sems1, sems2) = bufs
    rows = rows0
    @pl.loop(0, CHUNK)
    def _(i):
        rows[i] = jnp.zeros((LANES,), jnp.float32)

    abase = sid * NSEG

    @pl.loop(0, NSEG - CHUNK + 1, step=CHUNK)
    def _(i):
        pltpu.sync_copy(rows, acc.at[pl.ds(abase + i, CHUNK)])

    rem = NSEG % CHUNK
    pltpu.sync_copy(rows.at[pl.ds(0, rem)],
                    acc.at[pl.ds(abase + NSEG - rem, rem)])
    plsc.subcore_barrier()

    base = sid * RPT

    def _linears(r, srcv, dstv, ewv, sem):
        return [pltpu.async_copy(src_hbm.at[pl.ds(r, K)], srcv, sem),
                pltpu.async_copy(dst_hbm.at[pl.ds(r, K)], dstv, sem),
                pltpu.async_copy(ew_hbm.at[pl.ds(r, K)], ewv, sem)]

    def _gathers(srcv, rows, sem):
        return [pltpu.async_copy(table.at[srcv.at[j]],
                                 rows.at[pl.ds(j * 128, 128)], sem)
                for j in range(K)]

    def _scatters(rows, dstv, sem):
        return [pltpu.async_copy(rows.at[pl.ds(j * 128, 128)],
                                 acc.at[dstv.at[j]], sem, add=True)
                for j in range(K)]

    def _drain(cps):
        for cp in cps:
            cp.wait()

    @pl.loop(0, RPT, step=3 * K)
    def _(r):
        l0 = _linears(base + r, srcv0, dstv0, ewv0, seml0)
        l1 = _linears(base + r + K, srcv1, dstv1, ewv1, seml1)
        l2 = _linears(base + r + 2 * K, srcv2, dstv2, ewv2, seml2)
        _drain(l0)
        g0 = _gathers(srcv0, rows0, semg0)
        _drain(l1)
        g1 = _gathers(srcv1, rows1, semg1)
        _drain(g0)
        _scale_rows(rows0, ewv0)
        s0 = _scatters(rows0, dstv0, sems0)
        _drain(l2)
        g2 = _gathers(srcv2, rows2, semg2)
        _drain(g1)
        _scale_rows(rows1, ewv1)
        s1 = _scatters(rows1, dstv1, sems1)
        _drain(s0)
        _drain(g2)
        _scale_rows(rows2, ewv2)
        s2 = _scatters(rows2, dstv2, sems2)
        _drain(s1)
        _drain(s2)

    plsc.subcore_barrier()
    @pl.loop(0, NSEG - CHUNK + 1, step=CHUNK)
    def _(i):
        pltpu.sync_copy(acc.at[pl.ds(abase + i, CHUNK)], rows)
        pltpu.sync_copy(rows, out.at[pl.ds(abase + i, CHUNK)])

    pltpu.sync_copy(acc.at[pl.ds(abase + NSEG - rem, rem)],
                    rows.at[pl.ds(0, rem)])
    pltpu.sync_copy(rows.at[pl.ds(0, rem)],
                    out.at[pl.ds(abase + NSEG - rem, rem)])


def _agg_body(src_hbm, dst_hbm, ew_hbm, tlo_hbm, thi_hbm, olo_hbm, ohi_hbm,
              *bufs_acc):
    acc = bufs_acc[12]
    bufs = bufs_acc[:12] + bufs_acc[13:]
    cid = lax.axis_index("core")
    sid = lax.axis_index("subcore")

    @pl.when(cid == 0)
    def _():
        _agg_half(src_hbm, dst_hbm, ew_hbm, tlo_hbm, olo_hbm, bufs, acc, sid)

    @pl.when(cid == 1)
    def _():
        _agg_half(src_hbm, dst_hbm, ew_hbm, thi_hbm, ohi_hbm, bufs, acc, sid)


_agg_call = pl.kernel(
    _agg_body,
    out_type=[jax.ShapeDtypeStruct((NN, H), jnp.float32),
              jax.ShapeDtypeStruct((NN, H), jnp.float32)],
    mesh=_mesh,
    scratch_types=[
        pltpu.VMEM((K, 128), jnp.int32),
        pltpu.VMEM((K, 128), jnp.int32),
        pltpu.VMEM((K, 128), jnp.float32),
        pltpu.VMEM((CHUNK, H), jnp.float32),
        pltpu.VMEM((K, 128), jnp.int32),
        pltpu.VMEM((K, 128), jnp.int32),
        pltpu.VMEM((K, 128), jnp.float32),
        pltpu.VMEM((CHUNK, H), jnp.float32),
        pltpu.VMEM((K, 128), jnp.int32),
        pltpu.VMEM((K, 128), jnp.int32),
        pltpu.VMEM((K, 128), jnp.float32),
        pltpu.VMEM((CHUNK, H), jnp.float32),
        pltpu.VMEM_SHARED((NN, H), jnp.float32),
        pltpu.SemaphoreType.DMA,
        pltpu.SemaphoreType.DMA,
        pltpu.SemaphoreType.DMA,
        pltpu.SemaphoreType.DMA,
        pltpu.SemaphoreType.DMA,
        pltpu.SemaphoreType.DMA,
        pltpu.SemaphoreType.DMA,
        pltpu.SemaphoreType.DMA,
        pltpu.SemaphoreType.DMA,
    ],
    compiler_params=_sc_params,
)


def _mid_body(alo_ref, ahi_ref, tlo_ref, thi_ref, dis_ref, b1lo_ref, b1hi_ref,
              kll_ref, khl_ref, klh_ref, khh_ref, lo_ref, hi_ref):
    disp = dis_ref[...]
    hlo = jnp.maximum(disp * (alo_ref[...] + tlo_ref[...]) + b1lo_ref[...], 0.0)
    hhi = jnp.maximum(disp * (ahi_ref[...] + thi_ref[...]) + b1hi_ref[...], 0.0)
    h2lo = (jnp.dot(hlo, kll_ref[...], preferred_element_type=jnp.float32)
            + jnp.dot(hhi, khl_ref[...], preferred_element_type=jnp.float32))
    h2hi = (jnp.dot(hlo, klh_ref[...], preferred_element_type=jnp.float32)
            + jnp.dot(hhi, khh_ref[...], preferred_element_type=jnp.float32))
    lo_ref[...] = h2lo * disp
    hi_ref[...] = h2hi * disp


_mid_call = pl.pallas_call(
    _mid_body,
    grid=(NBLK,),
    in_specs=[
        pl.BlockSpec((BR, 128), lambda i: (i, 0)),
        pl.BlockSpec((BR, 128), lambda i: (i, 0)),
        pl.BlockSpec((BR, 128), lambda i: (i, 0)),
        pl.BlockSpec((BR, 128), lambda i: (i, 0)),
        pl.BlockSpec((BR, 128), lambda i: (i, 0)),
        pl.BlockSpec((1, 128), lambda i: (0, 0)),
        pl.BlockSpec((1, 128), lambda i: (0, 0)),
        pl.BlockSpec((128, 128), lambda i: (0, 0)),
        pl.BlockSpec((128, 128), lambda i: (0, 0)),
        pl.BlockSpec((128, 128), lambda i: (0, 0)),
        pl.BlockSpec((128, 128), lambda i: (0, 0)),
    ],
    out_specs=[
        pl.BlockSpec((BR, 128), lambda i: (i, 0)),
        pl.BlockSpec((BR, 128), lambda i: (i, 0)),
    ],
    out_shape=[jax.ShapeDtypeStruct((PROWS, 128), jnp.float32),
               jax.ShapeDtypeStruct((PROWS, 128), jnp.float32)],
)


def _fin_body(alo_ref, ahi_ref, tlo_ref, thi_ref, dis_ref, b2lo_ref, b2hi_ref,
              batch_ref, wo_ref, bo_ref, wb_ref, bb_ref,
              orange_ref, blue_ref, slo, shi, cnt):
    i = pl.program_id(0)

    @pl.when(i == 0)
    def _():
        slo[...] = jnp.zeros_like(slo)
        shi[...] = jnp.zeros_like(shi)
        cnt[...] = jnp.zeros_like(cnt)

    disp = dis_ref[...]
    hlo = jnp.maximum(disp * (alo_ref[...] + tlo_ref[...]) + b2lo_ref[...], 0.0)
    hhi = jnp.maximum(disp * (ahi_ref[...] + thi_ref[...]) + b2hi_ref[...], 0.0)
    bv = batch_ref[...]
    gids = lax.broadcasted_iota(jnp.int32, (G, BR), 0)
    for j in range(8):
        mask = (bv[j:j + 1, :] == gids).astype(jnp.float32)
        slo[...] += jnp.dot(mask, hlo[:, 16 * j:16 * j + 16],
                            preferred_element_type=jnp.float32)
        shi[...] += jnp.dot(mask, hhi[:, 16 * j:16 * j + 16],
                            preferred_element_type=jnp.float32)
        cnt[...] += jnp.sum(mask, axis=1, keepdims=True)

    @pl.when(i == NBLK - 1)
    def _():
        c = jnp.maximum(cnt[...], 1.0)
        glo = slo[...] / c
        ghi = shi[...] / c
        wo = wo_ref[...]
        wb = wb_ref[...]
        orange_ref[...] = jax.nn.sigmoid(
            jnp.dot(glo, wo[:16, :], preferred_element_type=jnp.float32)
            + jnp.dot(ghi, wo[16:, :], preferred_element_type=jnp.float32)
            + bo_ref[...])
        blue_ref[...] = jax.nn.sigmoid(
            jnp.dot(glo, wb[:16, :], preferred_element_type=jnp.float32)
            + jnp.dot(ghi, wb[16:, :], preferred_element_type=jnp.float32)
            + bb_ref[...])


_fin_call = pl.pallas_call(
    _fin_body,
    grid=(NBLK,),
    in_specs=[
        pl.BlockSpec((BR, 128), lambda i: (i, 0)),
        pl.BlockSpec((BR, 128), lambda i: (i, 0)),
        pl.BlockSpec((BR, 128), lambda i: (i, 0)),
        pl.BlockSpec((BR, 128), lambda i: (i, 0)),
        pl.BlockSpec((BR, 128), lambda i: (i, 0)),
        pl.BlockSpec((1, 128), lambda i: (0, 0)),
        pl.BlockSpec((1, 128), lambda i: (0, 0)),
        pl.BlockSpec((8, BR), lambda i: (0, i)),
        pl.BlockSpec((D, 1), lambda i: (0, 0)),
        pl.BlockSpec((1, 1), lambda i: (0, 0)),
        pl.BlockSpec((D, 1), lambda i: (0, 0)),
        pl.BlockSpec((1, 1), lambda i: (0, 0)),
    ],
    out_specs=[
        pl.BlockSpec((G, 1), lambda i: (0, 0)),
        pl.BlockSpec((G, 1), lambda i: (0, 0)),
    ],
    out_shape=[jax.ShapeDtypeStruct((G, 1), jnp.float32),
               jax.ShapeDtypeStruct((G, 1), jnp.float32)],
    scratch_shapes=[pltpu.VMEM((G, H), jnp.float32),
                    pltpu.VMEM((G, H), jnp.float32),
                    pltpu.VMEM((G, 1), jnp.float32)],
)


def kernel(x, edge_index, edge_weight, batch, W1, b1, W2, b2, Wo, bo, Wb, bb):
    pad = EP - E
    padidx = (jnp.arange(pad, dtype=jnp.int32) * 61) % N
    src = jnp.concatenate([edge_index[0], padidx]).reshape(ROWS, 128)
    dst = jnp.concatenate([edge_index[1], padidx]).reshape(ROWS, 128)
    ew = jnp.concatenate(
        [edge_weight, jnp.zeros((pad,), jnp.float32)]).reshape(ROWS, 128)

    xflat = jnp.concatenate(
        [x.reshape(-1), jnp.zeros(((NN - N) * 3,), jnp.float32)])
    w1flat = W1.reshape(-1)

    p0, p1 = _deg_call(dst, ew)
    tlo1, thi1, dis16 = _prep_call(xflat, w1flat, p0, p1)
    alo1, ahi1 = _agg_call(src, dst, ew, tlo1, thi1)

    disp = dis16.reshape(PROWS, 128)
    eye8 = jnp.eye(8, dtype=jnp.float32)
    kll = jnp.kron(eye8, W2[:16, :16])
    khl = jnp.kron(eye8, W2[16:, :16])
    klh = jnp.kron(eye8, W2[:16, 16:])
    khh = jnp.kron(eye8, W2[16:, 16:])
    b1lo = jnp.tile(b1[:16], 8).reshape(1, 128)
    b1hi = jnp.tile(b1[16:], 8).reshape(1, 128)
    b2lo = jnp.tile(b2[:16], 8).reshape(1, 128)
    b2hi = jnp.tile(b2[16:], 8).reshape(1, 128)

    lo2p, hi2p = _mid_call(alo1.reshape(PROWS, 128), ahi1.reshape(PROWS, 128),
                           tlo1.reshape(PROWS, 128), thi1.reshape(PROWS, 128),
                           disp, b1lo, b1hi, kll, khl, klh, khh)

    alo2, ahi2 = _agg_call(src, dst, ew,
                           lo2p.reshape(NN, H), hi2p.reshape(NN, H))

    batchp = jnp.concatenate(
        [batch, jnp.full((NN - N,), -1, jnp.int32)]).reshape(PROWS, 8).T

    orange, blue = _fin_call(alo2.reshape(PROWS, 128), ahi2.reshape(PROWS, 128),
                             lo2p, hi2p, disp, b2lo, b2hi, batchp,
                             Wo, bo.reshape(1, 1), Wb, bb.reshape(1, 1))
    return orange, blue


# speedup vs baseline: 1.3818x; 1.0073x over previous
import jax
import jax.numpy as jnp
from jax import lax
from jax.experimental import pallas as pl
from jax.experimental.pallas import tpu as pltpu
from jax.experimental.pallas import tpu_sc as plsc

N = 100000
E = 1600000
D = 32
H = 16
G = 64
LANES = 16
NC = 2
NS = 16
NW = NC * NS

EP = 1622016
ROWS = EP // 128
RPW = ROWS // NW
RPT = ROWS // NS
K = 4
CHUNK = K * 128
KD = 18

NN = 100352
NSEG = NN // NS
NPW = NN // NW
PC = 1568
PROWS = NN * H // 128

BR = 1792
NBLK = PROWS // BR

_mesh = plsc.VectorSubcoreMesh(core_axis_name="core", subcore_axis_name="subcore")
_sc_params = pltpu.CompilerParams(use_tc_tiling_on_sc=False)
_sc_params_nolayout = pltpu.CompilerParams(use_tc_tiling_on_sc=False,
                                           needs_layout_passes=False)

_DN = lax.GatherDimensionNumbers(
    offset_dims=(), collapsed_slice_dims=(0,), start_index_map=(0,))


def _bcast(vec, l):
    return lax.gather(vec, jnp.full((LANES, 1), l, jnp.int32), _DN, (1,),
                      mode=lax.GatherScatterMode.PROMISE_IN_BOUNDS)


def _rsqrt16(d):
    i = plsc.bitcast(d, jnp.int32)
    y = plsc.bitcast(jnp.int32(0x5F3759DF) - (i >> 1), jnp.float32)
    for _ in range(3):
        y = y * (1.5 - 0.5 * d * y * y)
    return y


def _deg_body(dst_hbm, ew_hbm, out0_hbm, out1_hbm, dstv0, ewv0, dstv1, ewv1,
              zbuf, acc, seml0, seml1, sems0, sems1):
    cid = lax.axis_index("core")
    sid = lax.axis_index("subcore")
    wid = sid * NC + cid

    @pl.loop(0, NSEG, step=LANES)
    def _(i):
        zbuf[pl.ds(i, LANES)] = jnp.zeros((LANES,), jnp.float32)

    pltpu.sync_copy(zbuf, acc.at[pl.ds(sid * NSEG, NSEG)])
    plsc.subcore_barrier()

    base = wid * RPW

    @pl.loop(0, RPW, step=2 * KD)
    def _(r):
        la = [pltpu.async_copy(dst_hbm.at[pl.ds(base + r, KD)], dstv0, seml0),
              pltpu.async_copy(ew_hbm.at[pl.ds(base + r, KD)], ewv0, seml0)]
        lb = [pltpu.async_copy(dst_hbm.at[pl.ds(base + r + KD, KD)], dstv1,
                               seml1),
              pltpu.async_copy(ew_hbm.at[pl.ds(base + r + KD, KD)], ewv1,
                               seml1)]
        for cp in la:
            cp.wait()
        sa = [pltpu.async_copy(ewv0.at[j], acc.at[dstv0.at[j]], sems0,
                               add=True) for j in range(KD)]
        for cp in lb:
            cp.wait()
        sb = [pltpu.async_copy(ewv1.at[j], acc.at[dstv1.at[j]], sems1,
                               add=True) for j in range(KD)]
        for cp in sa:
            cp.wait()
        for cp in sb:
            cp.wait()

    plsc.subcore_barrier()
    pltpu.sync_copy(acc.at[pl.ds(sid * NSEG, NSEG)], zbuf)

    @pl.when(cid == 0)
    def _():
        pltpu.sync_copy(zbuf, out0_hbm.at[pl.ds(sid * NSEG, NSEG)])

    @pl.when(cid == 1)
    def _():
        pltpu.sync_copy(zbuf, out1_hbm.at[pl.ds(sid * NSEG, NSEG)])


_deg_call = pl.kernel(
    _deg_body,
    out_type=[jax.ShapeDtypeStruct((NN,), jnp.float32),
              jax.ShapeDtypeStruct((NN,), jnp.float32)],
    mesh=_mesh,
    scratch_types=[
        pltpu.VMEM((KD, 128), jnp.int32),
        pltpu.VMEM((KD, 128), jnp.float32),
        pltpu.VMEM((KD, 128), jnp.int32),
        pltpu.VMEM((KD, 128), jnp.float32),
        pltpu.VMEM((NSEG,), jnp.float32),
        pltpu.VMEM_SHARED((NN,), jnp.float32),
        pltpu.SemaphoreType.DMA,
        pltpu.SemaphoreType.DMA,
        pltpu.SemaphoreType.DMA,
        pltpu.SemaphoreType.DMA,
    ],
    compiler_params=_sc_params,
)


def _prep_body(x_hbm, w1_hbm, p0_hbm, p1_hbm, tlo_hbm, thi_hbm, dis_hbm,
               xb, w1b, p0b, p1b, lob, hib, dsb):
    cid = lax.axis_index("core")
    sid = lax.axis_index("subcore")
    wid = sid * NC + cid
    nbase = wid * NPW

    pltpu.sync_copy(w1_hbm, w1b)
    w1v = [[w1b[pl.ds(32 * k + 16 * h_, LANES)] for h_ in range(2)]
           for k in range(3)]

    @pl.loop(0, NPW, step=PC)
    def _(c):
        node0 = nbase + c
        pltpu.sync_copy(x_hbm.at[pl.ds(node0 * 3, PC * 3)], xb)
        pltpu.sync_copy(p0_hbm.at[pl.ds(node0, PC)], p0b)
        pltpu.sync_copy(p1_hbm.at[pl.ds(node0, PC)], p1b)

        @pl.loop(0, PC, step=LANES)
        def _(g):
            d = p0b[pl.ds(g, LANES)] + p1b[pl.ds(g, LANES)] + 1.0
            y = _rsqrt16(d)
            xv = [xb[pl.ds(g * 3, LANES)],
                  xb[pl.ds(g * 3 + 16, LANES)],
                  xb[pl.ds(g * 3 + 32, LANES)]]
            for l in range(LANES):
                a = [_bcast(xv[(3 * l + k) // 16], (3 * l + k) % 16)
                     for k in range(3)]
                db = _bcast(y, l)
                rl = (a[0] * w1v[0][0] + a[1] * w1v[1][0]
                      + a[2] * w1v[2][0]) * db
                rh = (a[0] * w1v[0][1] + a[1] * w1v[1][1]
                      + a[2] * w1v[2][1]) * db
                lob[g + l] = rl
                hib[g + l] = rh
                dsb[g + l] = db

        pltpu.sync_copy(lob, tlo_hbm.at[pl.ds(node0, PC)])
        pltpu.sync_copy(hib, thi_hbm.at[pl.ds(node0, PC)])
        pltpu.sync_copy(dsb, dis_hbm.at[pl.ds(node0, PC)])


_prep_call = pl.kernel(
    _prep_body,
    out_type=[jax.ShapeDtypeStruct((NN, H), jnp.float32),
              jax.ShapeDtypeStruct((NN, H), jnp.float32),
              jax.ShapeDtypeStruct((NN, H), jnp.float32)],
    mesh=_mesh,
    scratch_types=[
        pltpu.VMEM((PC * 3,), jnp.float32),
        pltpu.VMEM((96,), jnp.float32),
        pltpu.VMEM((PC,), jnp.float32),
        pltpu.VMEM((PC,), jnp.float32),
        pltpu.VMEM((PC, H), jnp.float32),
        pltpu.VMEM((PC, H), jnp.float32),
        pltpu.VMEM((PC, H), jnp.float32),
    ],
    compiler_params=_sc_params_nolayout,
)


def _scale_rows(rows, ewv):
    for j in range(K):
        @pl.loop(0, 128, step=LANES)
        def _(e):
            w16 = ewv[j, pl.ds(e, LANES)]
            for l in range(LANES):
                idx = j * 128 + e + l
                rows[idx] = rows[idx] * _bcast(w16, l)


def _agg_half(src_hbm, dst_hbm, ew_hbm, table, out, bufs, acc, sid):
    (srcv0, dstv0, ewv0, rows0, srcv1, dstv1, ewv1, rows1,
     srcv2, dstv2, ewv2, rows2,
     seml0, seml1, seml2, semg0, semg1, semg2, sems0, sems1, sems2) = bufs
    rows = rows0
    @pl.loop(0, CHUNK)
    def _(i):
        rows[i] = jnp.zeros((LANES,), jnp.float32)

    abase = sid * NSEG
    rem = NSEG % CHUNK
    nz = NSEG // CHUNK
    zcps = [pltpu.async_copy(rows, acc.at[pl.ds(abase + i * CHUNK, CHUNK)],
                             seml0) for i in range(nz)]
    zcps.append(pltpu.async_copy(rows.at[pl.ds(0, rem)],
                                 acc.at[pl.ds(abase + NSEG - rem, rem)],
                                 seml0))
    for cp in zcps:
        cp.wait()
    plsc.subcore_barrier()

    base = sid * RPT

    def _linears(r, srcv, dstv, ewv, sem):
        return [pltpu.async_copy(src_hbm.at[pl.ds(r, K)], srcv, sem),
                pltpu.async_copy(dst_hbm.at[pl.ds(r, K)], dstv, sem),
                pltpu.async_copy(ew_hbm.at[pl.ds(r, K)], ewv, sem)]

    def _gathers(srcv, rows, sem):
        return [pltpu.async_copy(table.at[srcv.at[j]],
                                 rows.at[pl.ds(j * 128, 128)], sem)
                for j in range(K)]

    def _scatters(rows, dstv, sem):
        return [pltpu.async_copy(rows.at[pl.ds(j * 128, 128)],
                                 acc.at[dstv.at[j]], sem, add=True)
                for j in range(K)]

    def _drain(cps):
        for cp in cps:
            cp.wait()

    @pl.loop(0, RPT, step=3 * K)
    def _(r):
        l0 = _linears(base + r, srcv0, dstv0, ewv0, seml0)
        l1 = _linears(base + r + K, srcv1, dstv1, ewv1, seml1)
        l2 = _linears(base + r + 2 * K, srcv2, dstv2, ewv2, seml2)
        _drain(l0)
        g0 = _gathers(srcv0, rows0, semg0)
        _drain(l1)
        g1 = _gathers(srcv1, rows1, semg1)
        _drain(g0)
        _scale_rows(rows0, ewv0)
        s0 = _scatters(rows0, dstv0, sems0)
        _drain(l2)
        g2 = _gathers(srcv2, rows2, semg2)
        _drain(g1)
        _scale_rows(rows1, ewv1)
        s1 = _scatters(rows1, dstv1, sems1)
        _drain(s0)
        _drain(g2)
        _scale_rows(rows2, ewv2)
        s2 = _scatters(rows2, dstv2, sems2)
        _drain(s1)
        _drain(s2)

    plsc.subcore_barrier()
    allrows = [rows0, rows1, rows2]

    @pl.loop(0, NSEG - 3 * CHUNK + 1, step=3 * CHUNK)
    def _(i):
        ins = [pltpu.async_copy(acc.at[pl.ds(abase + i + b * CHUNK, CHUNK)],
                                allrows[b], seml0) for b in range(3)]
        outs = []
        for b in range(3):
            ins[b].wait()
            outs.append(pltpu.async_copy(
                allrows[b], out.at[pl.ds(abase + i + b * CHUNK, CHUNK)],
                seml1))
        for cp in outs:
            cp.wait()

    done = (NSEG // (3 * CHUNK)) * 3 * CHUNK
    tail = NSEG - done
    pltpu.sync_copy(acc.at[pl.ds(abase + done, tail)],
                    rows0.at[pl.ds(0, tail)])
    pltpu.sync_copy(rows0.at[pl.ds(0, tail)],
                    out.at[pl.ds(abase + done, tail)])


def _agg_body(src_hbm, dst_hbm, ew_hbm, tlo_hbm, thi_hbm, olo_hbm, ohi_hbm,
              *bufs_acc):
    acc = bufs_acc[12]
    bufs = bufs_acc[:12] + bufs_acc[13:]
    cid = lax.axis_index("core")
    sid = lax.axis_index("subcore")

    @pl.when(cid == 0)
    def _():
        _agg_half(src_hbm, dst_hbm, ew_hbm, tlo_hbm, olo_hbm, bufs, acc, sid)

    @pl.when(cid == 1)
    def _():
        _agg_half(src_hbm, dst_hbm, ew_hbm, thi_hbm, ohi_hbm, bufs, acc, sid)


_agg_call = pl.kernel(
    _agg_body,
    out_type=[jax.ShapeDtypeStruct((NN, H), jnp.float32),
              jax.ShapeDtypeStruct((NN, H), jnp.float32)],
    mesh=_mesh,
    scratch_types=[
        pltpu.VMEM((K, 128), jnp.int32),
        pltpu.VMEM((K, 128), jnp.int32),
        pltpu.VMEM((K, 128), jnp.float32),
        pltpu.VMEM((CHUNK, H), jnp.float32),
        pltpu.VMEM((K, 128), jnp.int32),
        pltpu.VMEM((K, 128), jnp.int32),
        pltpu.VMEM((K, 128), jnp.float32),
        pltpu.VMEM((CHUNK, H), jnp.float32),
        pltpu.VMEM((K, 128), jnp.int32),
        pltpu.VMEM((K, 128), jnp.int32),
        pltpu.VMEM((K, 128), jnp.float32),
        pltpu.VMEM((CHUNK, H), jnp.float32),
        pltpu.VMEM_SHARED((NN, H), jnp.float32),
        pltpu.SemaphoreType.DMA,
        pltpu.SemaphoreType.DMA,
        pltpu.SemaphoreType.DMA,
        pltpu.SemaphoreType.DMA,
        pltpu.SemaphoreType.DMA,
        pltpu.SemaphoreType.DMA,
        pltpu.SemaphoreType.DMA,
        pltpu.SemaphoreType.DMA,
        pltpu.SemaphoreType.DMA,
    ],
    compiler_params=_sc_params,
)


def _mid_body(alo_ref, ahi_ref, tlo_ref, thi_ref, dis_ref, b1lo_ref, b1hi_ref,
              kll_ref, khl_ref, klh_ref, khh_ref, lo_ref, hi_ref):
    disp = dis_ref[...]
    hlo = jnp.maximum(disp * (alo_ref[...] + tlo_ref[...]) + b1lo_ref[...], 0.0)
    hhi = jnp.maximum(disp * (ahi_ref[...] + thi_ref[...]) + b1hi_ref[...], 0.0)
    h2lo = (jnp.dot(hlo, kll_ref[...], preferred_element_type=jnp.float32)
            + jnp.dot(hhi, khl_ref[...], preferred_element_type=jnp.float32))
    h2hi = (jnp.dot(hlo, klh_ref[...], preferred_element_type=jnp.float32)
            + jnp.dot(hhi, khh_ref[...], preferred_element_type=jnp.float32))
    lo_ref[...] = h2lo * disp
    hi_ref[...] = h2hi * disp


_mid_call = pl.pallas_call(
    _mid_body,
    grid=(NBLK,),
    in_specs=[
        pl.BlockSpec((BR, 128), lambda i: (i, 0)),
        pl.BlockSpec((BR, 128), lambda i: (i, 0)),
        pl.BlockSpec((BR, 128), lambda i: (i, 0)),
        pl.BlockSpec((BR, 128), lambda i: (i, 0)),
        pl.BlockSpec((BR, 128), lambda i: (i, 0)),
        pl.BlockSpec((1, 128), lambda i: (0, 0)),
        pl.BlockSpec((1, 128), lambda i: (0, 0)),
        pl.BlockSpec((128, 128), lambda i: (0, 0)),
        pl.BlockSpec((128, 128), lambda i: (0, 0)),
        pl.BlockSpec((128, 128), lambda i: (0, 0)),
        pl.BlockSpec((128, 128), lambda i: (0, 0)),
    ],
    out_specs=[
        pl.BlockSpec((BR, 128), lambda i: (i, 0)),
        pl.BlockSpec((BR, 128), lambda i: (i, 0)),
    ],
    out_shape=[jax.ShapeDtypeStruct((PROWS, 128), jnp.float32),
               jax.ShapeDtypeStruct((PROWS, 128), jnp.float32)],
)


def _fin_body(alo_ref, ahi_ref, tlo_ref, thi_ref, dis_ref, b2lo_ref, b2hi_ref,
              batch_ref, wo_ref, bo_ref, wb_ref, bb_ref,
              orange_ref, blue_ref, slo, shi, cnt):
    i = pl.program_id(0)

    @pl.when(i == 0)
    def _():
        slo[...] = jnp.zeros_like(slo)
        shi[...] = jnp.zeros_like(shi)
        cnt[...] = jnp.zeros_like(cnt)

    disp = dis_ref[...]
    hlo = jnp.maximum(disp * (alo_ref[...] + tlo_ref[...]) + b2lo_ref[...], 0.0)
    hhi = jnp.maximum(disp * (ahi_ref[...] + thi_ref[...]) + b2hi_ref[...], 0.0)
    bv = batch_ref[...]
    gids = lax.broadcasted_iota(jnp.int32, (G, BR), 0)
    for j in range(8):
        mask = (bv[j:j + 1, :] == gids).astype(jnp.float32)
        slo[...] += jnp.dot(mask, hlo[:, 16 * j:16 * j + 16],
                            preferred_element_type=jnp.float32)
        shi[...] += jnp.dot(mask, hhi[:, 16 * j:16 * j + 16],
                            preferred_element_type=jnp.float32)
        cnt[...] += jnp.sum(mask, axis=1, keepdims=True)

    @pl.when(i == NBLK - 1)
    def _():
        c = jnp.maximum(cnt[...], 1.0)
        glo = slo[...] / c
        ghi = shi[...] / c
        wo = wo_ref[...]
        wb = wb_ref[...]
        orange_ref[...] = jax.nn.sigmoid(
            jnp.dot(glo, wo[:16, :], preferred_element_type=jnp.float32)
            + jnp.dot(ghi, wo[16:, :], preferred_element_type=jnp.float32)
            + bo_ref[...])
        blue_ref[...] = jax.nn.sigmoid(
            jnp.dot(glo, wb[:16, :], preferred_element_type=jnp.float32)
            + jnp.dot(ghi, wb[16:, :], preferred_element_type=jnp.float32)
            + bb_ref[...])


_fin_call = pl.pallas_call(
    _fin_body,
    grid=(NBLK,),
    in_specs=[
        pl.BlockSpec((BR, 128), lambda i: (i, 0)),
        pl.BlockSpec((BR, 128), lambda i: (i, 0)),
        pl.BlockSpec((BR, 128), lambda i: (i, 0)),
        pl.BlockSpec((BR, 128), lambda i: (i, 0)),
        pl.BlockSpec((BR, 128), lambda i: (i, 0)),
        pl.BlockSpec((1, 128), lambda i: (0, 0)),
        pl.BlockSpec((1, 128), lambda i: (0, 0)),
        pl.BlockSpec((8, BR), lambda i: (0, i)),
        pl.BlockSpec((D, 1), lambda i: (0, 0)),
        pl.BlockSpec((1, 1), lambda i: (0, 0)),
        pl.BlockSpec((D, 1), lambda i: (0, 0)),
        pl.BlockSpec((1, 1), lambda i: (0, 0)),
    ],
    out_specs=[
        pl.BlockSpec((G, 1), lambda i: (0, 0)),
        pl.BlockSpec((G, 1), lambda i: (0, 0)),
    ],
    out_shape=[jax.ShapeDtypeStruct((G, 1), jnp.float32),
               jax.ShapeDtypeStruct((G, 1), jnp.float32)],
    scratch_shapes=[pltpu.VMEM((G, H), jnp.float32),
                    pltpu.VMEM((G, H), jnp.float32),
                    pltpu.VMEM((G, 1), jnp.float32)],
)


def kernel(x, edge_index, edge_weight, batch, W1, b1, W2, b2, Wo, bo, Wb, bb):
    pad = EP - E
    padidx = (jnp.arange(pad, dtype=jnp.int32) * 61) % N
    src = jnp.concatenate([edge_index[0], padidx]).reshape(ROWS, 128)
    dst = jnp.concatenate([edge_index[1], padidx]).reshape(ROWS, 128)
    ew = jnp.concatenate(
        [edge_weight, jnp.zeros((pad,), jnp.float32)]).reshape(ROWS, 128)

    xflat = jnp.concatenate(
        [x.reshape(-1), jnp.zeros(((NN - N) * 3,), jnp.float32)])
    w1flat = W1.reshape(-1)

    p0, p1 = _deg_call(dst, ew)
    tlo1, thi1, dis16 = _prep_call(xflat, w1flat, p0, p1)
    alo1, ahi1 = _agg_call(src, dst, ew, tlo1, thi1)

    disp = dis16.reshape(PROWS, 128)
    eye8 = jnp.eye(8, dtype=jnp.float32)
    kll = jnp.kron(eye8, W2[:16, :16])
    khl = jnp.kron(eye8, W2[16:, :16])
    klh = jnp.kron(eye8, W2[:16, 16:])
    khh = jnp.kron(eye8, W2[16:, 16:])
    b1lo = jnp.tile(b1[:16], 8).reshape(1, 128)
    b1hi = jnp.tile(b1[16:], 8).reshape(1, 128)
    b2lo = jnp.tile(b2[:16], 8).reshape(1, 128)
    b2hi = jnp.tile(b2[16:], 8).reshape(1, 128)

    lo2p, hi2p = _mid_call(alo1.reshape(PROWS, 128), ahi1.reshape(PROWS, 128),
                           tlo1.reshape(PROWS, 128), thi1.reshape(PROWS, 128),
                           disp, b1lo, b1hi, kll, khl, klh, khh)

    alo2, ahi2 = _agg_call(src, dst, ew,
                           lo2p.reshape(NN, H), hi2p.reshape(NN, H))

    batchp = jnp.concatenate(
        [batch, jnp.full((NN - N,), -1, jnp.int32)]).reshape(PROWS, 8).T

    orange, blue = _fin_call(alo2.reshape(PROWS, 128), ahi2.reshape(PROWS, 128),
                             lo2p, hi2p, disp, b2lo, b2hi, batchp,
                             Wo, bo.reshape(1, 1), Wb, bb.reshape(1, 1))
    return orange, blue


# speedup vs baseline: 1.4391x; 1.0415x over previous
import jax
import jax.numpy as jnp
from jax import lax
from jax.experimental import pallas as pl
from jax.experimental.pallas import tpu as pltpu
from jax.experimental.pallas import tpu_sc as plsc

N = 100000
E = 1600000
D = 32
H = 16
G = 64
LANES = 16
NC = 2
NS = 16
NW = NC * NS

EP = 1622016
ROWS = EP // 128
RPW = ROWS // NW
RPT = ROWS // NS
K = 3
CHUNK = K * 128
KD = 18

NN = 100352
NSEG = NN // NS
NPW = NN // NW
PC = 1568
PROWS = NN * H // 128

BR = 1792
NBLK = PROWS // BR

_mesh = plsc.VectorSubcoreMesh(core_axis_name="core", subcore_axis_name="subcore")
_sc_params = pltpu.CompilerParams(use_tc_tiling_on_sc=False)
_sc_params_nolayout = pltpu.CompilerParams(use_tc_tiling_on_sc=False,
                                           needs_layout_passes=False)

_DN = lax.GatherDimensionNumbers(
    offset_dims=(), collapsed_slice_dims=(0,), start_index_map=(0,))


def _bcast(vec, l):
    return lax.gather(vec, jnp.full((LANES, 1), l, jnp.int32), _DN, (1,),
                      mode=lax.GatherScatterMode.PROMISE_IN_BOUNDS)


def _rsqrt16(d):
    i = plsc.bitcast(d, jnp.int32)
    y = plsc.bitcast(jnp.int32(0x5F3759DF) - (i >> 1), jnp.float32)
    for _ in range(3):
        y = y * (1.5 - 0.5 * d * y * y)
    return y


def _deg_body(dst_hbm, ew_hbm, out0_hbm, out1_hbm, dstv0, ewv0, dstv1, ewv1,
              zbuf, acc, seml0, seml1, sems0, sems1):
    cid = lax.axis_index("core")
    sid = lax.axis_index("subcore")
    wid = sid * NC + cid

    @pl.loop(0, NSEG, step=LANES)
    def _(i):
        zbuf[pl.ds(i, LANES)] = jnp.zeros((LANES,), jnp.float32)

    pltpu.sync_copy(zbuf, acc.at[pl.ds(sid * NSEG, NSEG)])
    plsc.subcore_barrier()

    base = wid * RPW

    @pl.loop(0, RPW, step=2 * KD)
    def _(r):
        la = [pltpu.async_copy(dst_hbm.at[pl.ds(base + r, KD)], dstv0, seml0),
              pltpu.async_copy(ew_hbm.at[pl.ds(base + r, KD)], ewv0, seml0)]
        lb = [pltpu.async_copy(dst_hbm.at[pl.ds(base + r + KD, KD)], dstv1,
                               seml1),
              pltpu.async_copy(ew_hbm.at[pl.ds(base + r + KD, KD)], ewv1,
                               seml1)]
        for cp in la:
            cp.wait()
        sa = [pltpu.async_copy(ewv0.at[j], acc.at[dstv0.at[j]], sems0,
                               add=True) for j in range(KD)]
        for cp in lb:
            cp.wait()
        sb = [pltpu.async_copy(ewv1.at[j], acc.at[dstv1.at[j]], sems1,
                               add=True) for j in range(KD)]
        for cp in sa:
            cp.wait()
        for cp in sb:
            cp.wait()

    plsc.subcore_barrier()
    pltpu.sync_copy(acc.at[pl.ds(sid * NSEG, NSEG)], zbuf)

    @pl.when(cid == 0)
    def _():
        pltpu.sync_copy(zbuf, out0_hbm.at[pl.ds(sid * NSEG, NSEG)])

    @pl.when(cid == 1)
    def _():
        pltpu.sync_copy(zbuf, out1_hbm.at[pl.ds(sid * NSEG, NSEG)])


_deg_call = pl.kernel(
    _deg_body,
    out_type=[jax.ShapeDtypeStruct((NN,), jnp.float32),
              jax.ShapeDtypeStruct((NN,), jnp.float32)],
    mesh=_mesh,
    scratch_types=[
        pltpu.VMEM((KD, 128), jnp.int32),
        pltpu.VMEM((KD, 128), jnp.float32),
        pltpu.VMEM((KD, 128), jnp.int32),
        pltpu.VMEM((KD, 128), jnp.float32),
        pltpu.VMEM((NSEG,), jnp.float32),
        pltpu.VMEM_SHARED((NN,), jnp.float32),
        pltpu.SemaphoreType.DMA,
        pltpu.SemaphoreType.DMA,
        pltpu.SemaphoreType.DMA,
        pltpu.SemaphoreType.DMA,
    ],
    compiler_params=_sc_params,
)


def _prep_body(x_hbm, w1_hbm, p0_hbm, p1_hbm, tlo_hbm, thi_hbm, dis_hbm,
               xb, w1b, p0b, p1b, lob, hib, dsb):
    cid = lax.axis_index("core")
    sid = lax.axis_index("subcore")
    wid = sid * NC + cid
    nbase = wid * NPW

    pltpu.sync_copy(w1_hbm, w1b)
    w1v = [[w1b[pl.ds(32 * k + 16 * h_, LANES)] for h_ in range(2)]
           for k in range(3)]

    @pl.loop(0, NPW, step=PC)
    def _(c):
        node0 = nbase + c
        pltpu.sync_copy(x_hbm.at[pl.ds(node0 * 3, PC * 3)], xb)
        pltpu.sync_copy(p0_hbm.at[pl.ds(node0, PC)], p0b)
        pltpu.sync_copy(p1_hbm.at[pl.ds(node0, PC)], p1b)

        @pl.loop(0, PC, step=LANES)
        def _(g):
            d = p0b[pl.ds(g, LANES)] + p1b[pl.ds(g, LANES)] + 1.0
            y = _rsqrt16(d)
            xv = [xb[pl.ds(g * 3, LANES)],
                  xb[pl.ds(g * 3 + 16, LANES)],
                  xb[pl.ds(g * 3 + 32, LANES)]]
            for l in range(LANES):
                a = [_bcast(xv[(3 * l + k) // 16], (3 * l + k) % 16)
                     for k in range(3)]
                db = _bcast(y, l)
                rl = (a[0] * w1v[0][0] + a[1] * w1v[1][0]
                      + a[2] * w1v[2][0]) * db
                rh = (a[0] * w1v[0][1] + a[1] * w1v[1][1]
                      + a[2] * w1v[2][1]) * db
                lob[g + l] = rl
                hib[g + l] = rh
                dsb[g + l] = db

        pltpu.sync_copy(lob, tlo_hbm.at[pl.ds(node0, PC)])
        pltpu.sync_copy(hib, thi_hbm.at[pl.ds(node0, PC)])
        pltpu.sync_copy(dsb, dis_hbm.at[pl.ds(node0, PC)])


_prep_call = pl.kernel(
    _prep_body,
    out_type=[jax.ShapeDtypeStruct((NN, H), jnp.float32),
              jax.ShapeDtypeStruct((NN, H), jnp.float32),
              jax.ShapeDtypeStruct((NN, H), jnp.float32)],
    mesh=_mesh,
    scratch_types=[
        pltpu.VMEM((PC * 3,), jnp.float32),
        pltpu.VMEM((96,), jnp.float32),
        pltpu.VMEM((PC,), jnp.float32),
        pltpu.VMEM((PC,), jnp.float32),
        pltpu.VMEM((PC, H), jnp.float32),
        pltpu.VMEM((PC, H), jnp.float32),
        pltpu.VMEM((PC, H), jnp.float32),
    ],
    compiler_params=_sc_params_nolayout,
)


def _scale_rows(rows, ewv):
    for j in range(K):
        @pl.loop(0, 128, step=LANES)
        def _(e):
            w16 = ewv[j, pl.ds(e, LANES)]
            for l in range(LANES):
                idx = j * 128 + e + l
                rows[idx] = rows[idx] * _bcast(w16, l)


def _agg_half(src_hbm, dst_hbm, ew_hbm, table, out, bufs, acc, sid):
    (srcv0, dstv0, ewv0, rows0, srcv1, dstv1, ewv1, rows1,
     srcv2, dstv2, ewv2, rows2, srcv3, dstv3, ewv3, rows3,
     seml0, seml1, seml2, seml3, semg0, semg1, semg2, semg3,
     sems0, sems1, sems2, sems3) = bufs
    rows = rows0
    @pl.loop(0, CHUNK)
    def _(i):
        rows[i] = jnp.zeros((LANES,), jnp.float32)

    abase = sid * NSEG
    rem = NSEG % CHUNK
    nz = NSEG // CHUNK
    zcps = [pltpu.async_copy(rows, acc.at[pl.ds(abase + i * CHUNK, CHUNK)],
                             seml0) for i in range(nz)]
    zcps.append(pltpu.async_copy(rows.at[pl.ds(0, rem)],
                                 acc.at[pl.ds(abase + NSEG - rem, rem)],
                                 seml0))
    for cp in zcps:
        cp.wait()
    plsc.subcore_barrier()

    base = sid * RPT

    def _linears(r, srcv, dstv, ewv, sem):
        return [pltpu.async_copy(src_hbm.at[pl.ds(r, K)], srcv, sem),
                pltpu.async_copy(dst_hbm.at[pl.ds(r, K)], dstv, sem),
                pltpu.async_copy(ew_hbm.at[pl.ds(r, K)], ewv, sem)]

    def _gathers(srcv, rows, sem):
        return [pltpu.async_copy(table.at[srcv.at[j]],
                                 rows.at[pl.ds(j * 128, 128)], sem)
                for j in range(K)]

    def _scatters(rows, dstv, sem):
        return [pltpu.async_copy(rows.at[pl.ds(j * 128, 128)],
                                 acc.at[dstv.at[j]], sem, add=True)
                for j in range(K)]

    def _drain(cps):
        for cp in cps:
            cp.wait()

    @pl.loop(0, RPT, step=4 * K)
    def _(r):
        l0 = _linears(base + r, srcv0, dstv0, ewv0, seml0)
        l1 = _linears(base + r + K, srcv1, dstv1, ewv1, seml1)
        l2 = _linears(base + r + 2 * K, srcv2, dstv2, ewv2, seml2)
        l3 = _linears(base + r + 3 * K, srcv3, dstv3, ewv3, seml3)
        _drain(l0)
        g0 = _gathers(srcv0, rows0, semg0)
        _drain(l1)
        g1 = _gathers(srcv1, rows1, semg1)
        _drain(g0)
        _scale_rows(rows0, ewv0)
        s0 = _scatters(rows0, dstv0, sems0)
        _drain(l2)
        g2 = _gathers(srcv2, rows2, semg2)
        _drain(g1)
        _scale_rows(rows1, ewv1)
        s1 = _scatters(rows1, dstv1, sems1)
        _drain(l3)
        g3 = _gathers(srcv3, rows3, semg3)
        _drain(s0)
        _drain(g2)
        _scale_rows(rows2, ewv2)
        s2 = _scatters(rows2, dstv2, sems2)
        _drain(s1)
        _drain(g3)
        _scale_rows(rows3, ewv3)
        s3 = _scatters(rows3, dstv3, sems3)
        _drain(s2)
        _drain(s3)

    plsc.subcore_barrier()
    allrows = [rows0, rows1, rows2, rows3]

    @pl.loop(0, NSEG - 4 * CHUNK + 1, step=4 * CHUNK)
    def _(i):
        ins = [pltpu.async_copy(acc.at[pl.ds(abase + i + b * CHUNK, CHUNK)],
                                allrows[b], seml0) for b in range(4)]
        outs = []
        for b in range(4):
            ins[b].wait()
            outs.append(pltpu.async_copy(
                allrows[b], out.at[pl.ds(abase + i + b * CHUNK, CHUNK)],
                seml1))
        for cp in outs:
            cp.wait()

    done = (NSEG // (4 * CHUNK)) * 4 * CHUNK
    tail = NSEG - done
    pltpu.sync_copy(acc.at[pl.ds(abase + done, tail)],
                    rows0.at[pl.ds(0, tail)])
    pltpu.sync_copy(rows0.at[pl.ds(0, tail)],
                    out.at[pl.ds(abase + done, tail)])


def _agg_body(src_hbm, dst_hbm, ew_hbm, tlo_hbm, thi_hbm, olo_hbm, ohi_hbm,
              *bufs_acc):
    acc = bufs_acc[16]
    bufs = bufs_acc[:16] + bufs_acc[17:]
    cid = lax.axis_index("core")
    sid = lax.axis_index("subcore")

    @pl.when(cid == 0)
    def _():
        _agg_half(src_hbm, dst_hbm, ew_hbm, tlo_hbm, olo_hbm, bufs, acc, sid)

    @pl.when(cid == 1)
    def _():
        _agg_half(src_hbm, dst_hbm, ew_hbm, thi_hbm, ohi_hbm, bufs, acc, sid)


_agg_call = pl.kernel(
    _agg_body,
    out_type=[jax.ShapeDtypeStruct((NN, H), jnp.float32),
              jax.ShapeDtypeStruct((NN, H), jnp.float32)],
    mesh=_mesh,
    scratch_types=[
        pltpu.VMEM((K, 128), jnp.int32),
        pltpu.VMEM((K, 128), jnp.int32),
        pltpu.VMEM((K, 128), jnp.float32),
        pltpu.VMEM((CHUNK, H), jnp.float32),
        pltpu.VMEM((K, 128), jnp.int32),
        pltpu.VMEM((K, 128), jnp.int32),
        pltpu.VMEM((K, 128), jnp.float32),
        pltpu.VMEM((CHUNK, H), jnp.float32),
        pltpu.VMEM((K, 128), jnp.int32),
        pltpu.VMEM((K, 128), jnp.int32),
        pltpu.VMEM((K, 128), jnp.float32),
        pltpu.VMEM((CHUNK, H), jnp.float32),
        pltpu.VMEM((K, 128), jnp.int32),
        pltpu.VMEM((K, 128), jnp.int32),
        pltpu.VMEM((K, 128), jnp.float32),
        pltpu.VMEM((CHUNK, H), jnp.float32),
        pltpu.VMEM_SHARED((NN, H), jnp.float32),
        pltpu.SemaphoreType.DMA,
        pltpu.SemaphoreType.DMA,
        pltpu.SemaphoreType.DMA,
        pltpu.SemaphoreType.DMA,
        pltpu.SemaphoreType.DMA,
        pltpu.SemaphoreType.DMA,
        pltpu.SemaphoreType.DMA,
        pltpu.SemaphoreType.DMA,
        pltpu.SemaphoreType.DMA,
        pltpu.SemaphoreType.DMA,
        pltpu.SemaphoreType.DMA,
        pltpu.SemaphoreType.DMA,
    ],
    compiler_params=_sc_params,
)


def _mid_body(alo_ref, ahi_ref, tlo_ref, thi_ref, dis_ref, b1lo_ref, b1hi_ref,
              kll_ref, khl_ref, klh_ref, khh_ref, lo_ref, hi_ref):
    disp = dis_ref[...]
    hlo = jnp.maximum(disp * (alo_ref[...] + tlo_ref[...]) + b1lo_ref[...], 0.0)
    hhi = jnp.maximum(disp * (ahi_ref[...] + thi_ref[...]) + b1hi_ref[...], 0.0)
    h2lo = (jnp.dot(hlo, kll_ref[...], preferred_element_type=jnp.float32)
            + jnp.dot(hhi, khl_ref[...], preferred_element_type=jnp.float32))
    h2hi = (jnp.dot(hlo, klh_ref[...], preferred_element_type=jnp.float32)
            + jnp.dot(hhi, khh_ref[...], preferred_element_type=jnp.float32))
    lo_ref[...] = h2lo * disp
    hi_ref[...] = h2hi * disp


_mid_call = pl.pallas_call(
    _mid_body,
    grid=(NBLK,),
    in_specs=[
        pl.BlockSpec((BR, 128), lambda i: (i, 0)),
        pl.BlockSpec((BR, 128), lambda i: (i, 0)),
        pl.BlockSpec((BR, 128), lambda i: (i, 0)),
        pl.BlockSpec((BR, 128), lambda i: (i, 0)),
        pl.BlockSpec((BR, 128), lambda i: (i, 0)),
        pl.BlockSpec((1, 128), lambda i: (0, 0)),
        pl.BlockSpec((1, 128), lambda i: (0, 0)),
        pl.BlockSpec((128, 128), lambda i: (0, 0)),
        pl.BlockSpec((128, 128), lambda i: (0, 0)),
        pl.BlockSpec((128, 128), lambda i: (0, 0)),
        pl.BlockSpec((128, 128), lambda i: (0, 0)),
    ],
    out_specs=[
        pl.BlockSpec((BR, 128), lambda i: (i, 0)),
        pl.BlockSpec((BR, 128), lambda i: (i, 0)),
    ],
    out_shape=[jax.ShapeDtypeStruct((PROWS, 128), jnp.float32),
               jax.ShapeDtypeStruct((PROWS, 128), jnp.float32)],
)


def _fin_body(alo_ref, ahi_ref, tlo_ref, thi_ref, dis_ref, b2lo_ref, b2hi_ref,
              batch_ref, wo_ref, bo_ref, wb_ref, bb_ref,
              orange_ref, blue_ref, slo, shi, cnt):
    i = pl.program_id(0)

    @pl.when(i == 0)
    def _():
        slo[...] = jnp.zeros_like(slo)
        shi[...] = jnp.zeros_like(shi)
        cnt[...] = jnp.zeros_like(cnt)

    disp = dis_ref[...]
    hlo = jnp.maximum(disp * (alo_ref[...] + tlo_ref[...]) + b2lo_ref[...], 0.0)
    hhi = jnp.maximum(disp * (ahi_ref[...] + thi_ref[...]) + b2hi_ref[...], 0.0)
    bv = batch_ref[...]
    gids = lax.broadcasted_iota(jnp.int32, (G, BR), 0)
    for j in range(8):
        mask = (bv[j:j + 1, :] == gids).astype(jnp.float32)
        slo[...] += jnp.dot(mask, hlo[:, 16 * j:16 * j + 16],
                            preferred_element_type=jnp.float32)
        shi[...] += jnp.dot(mask, hhi[:, 16 * j:16 * j + 16],
                            preferred_element_type=jnp.float32)
        cnt[...] += jnp.sum(mask, axis=1, keepdims=True)

    @pl.when(i == NBLK - 1)
    def _():
        c = jnp.maximum(cnt[...], 1.0)
        glo = slo[...] / c
        ghi = shi[...] / c
        wo = wo_ref[...]
        wb = wb_ref[...]
        orange_ref[...] = jax.nn.sigmoid(
            jnp.dot(glo, wo[:16, :], preferred_element_type=jnp.float32)
            + jnp.dot(ghi, wo[16:, :], preferred_element_type=jnp.float32)
            + bo_ref[...])
        blue_ref[...] = jax.nn.sigmoid(
            jnp.dot(glo, wb[:16, :], preferred_element_type=jnp.float32)
            + jnp.dot(ghi, wb[16:, :], preferred_element_type=jnp.float32)
            + bb_ref[...])


_fin_call = pl.pallas_call(
    _fin_body,
    grid=(NBLK,),
    in_specs=[
        pl.BlockSpec((BR, 128), lambda i: (i, 0)),
        pl.BlockSpec((BR, 128), lambda i: (i, 0)),
        pl.BlockSpec((BR, 128), lambda i: (i, 0)),
        pl.BlockSpec((BR, 128), lambda i: (i, 0)),
        pl.BlockSpec((BR, 128), lambda i: (i, 0)),
        pl.BlockSpec((1, 128), lambda i: (0, 0)),
        pl.BlockSpec((1, 128), lambda i: (0, 0)),
        pl.BlockSpec((8, BR), lambda i: (0, i)),
        pl.BlockSpec((D, 1), lambda i: (0, 0)),
        pl.BlockSpec((1, 1), lambda i: (0, 0)),
        pl.BlockSpec((D, 1), lambda i: (0, 0)),
        pl.BlockSpec((1, 1), lambda i: (0, 0)),
    ],
    out_specs=[
        pl.BlockSpec((G, 1), lambda i: (0, 0)),
        pl.BlockSpec((G, 1), lambda i: (0, 0)),
    ],
    out_shape=[jax.ShapeDtypeStruct((G, 1), jnp.float32),
               jax.ShapeDtypeStruct((G, 1), jnp.float32)],
    scratch_shapes=[pltpu.VMEM((G, H), jnp.float32),
                    pltpu.VMEM((G, H), jnp.float32),
                    pltpu.VMEM((G, 1), jnp.float32)],
)


def kernel(x, edge_index, edge_weight, batch, W1, b1, W2, b2, Wo, bo, Wb, bb):
    pad = EP - E
    padidx = (jnp.arange(pad, dtype=jnp.int32) * 61) % N
    src = jnp.concatenate([edge_index[0], padidx]).reshape(ROWS, 128)
    dst = jnp.concatenate([edge_index[1], padidx]).reshape(ROWS, 128)
    ew = jnp.concatenate(
        [edge_weight, jnp.zeros((pad,), jnp.float32)]).reshape(ROWS, 128)

    xflat = jnp.concatenate(
        [x.reshape(-1), jnp.zeros(((NN - N) * 3,), jnp.float32)])
    w1flat = W1.reshape(-1)

    p0, p1 = _deg_call(dst, ew)
    tlo1, thi1, dis16 = _prep_call(xflat, w1flat, p0, p1)
    alo1, ahi1 = _agg_call(src, dst, ew, tlo1, thi1)

    disp = dis16.reshape(PROWS, 128)
    eye8 = jnp.eye(8, dtype=jnp.float32)
    kll = jnp.kron(eye8, W2[:16, :16])
    khl = jnp.kron(eye8, W2[16:, :16])
    klh = jnp.kron(eye8, W2[:16, 16:])
    khh = jnp.kron(eye8, W2[16:, 16:])
    b1lo = jnp.tile(b1[:16], 8).reshape(1, 128)
    b1hi = jnp.tile(b1[16:], 8).reshape(1, 128)
    b2lo = jnp.tile(b2[:16], 8).reshape(1, 128)
    b2hi = jnp.tile(b2[16:], 8).reshape(1, 128)

    lo2p, hi2p = _mid_call(alo1.reshape(PROWS, 128), ahi1.reshape(PROWS, 128),
                           tlo1.reshape(PROWS, 128), thi1.reshape(PROWS, 128),
                           disp, b1lo, b1hi, kll, khl, klh, khh)

    alo2, ahi2 = _agg_call(src, dst, ew,
                           lo2p.reshape(NN, H), hi2p.reshape(NN, H))

    batchp = jnp.concatenate(
        [batch, jnp.full((NN - N,), -1, jnp.int32)]).reshape(PROWS, 8).T

    orange, blue = _fin_call(alo2.reshape(PROWS, 128), ahi2.reshape(PROWS, 128),
                             lo2p, hi2p, disp, b2lo, b2hi, batchp,
                             Wo, bo.reshape(1, 1), Wb, bb.reshape(1, 1))
    return orange, blue
